# Initial kernel scaffold; baseline (speedup 1.0000x reference)
import jax
import jax.numpy as jnp
from jax.experimental import pallas as pl


def kernel(x, x_orth, beta, alpha, idx, W1, b1, W2, b2, Wg1, Wg2):
    raise NotImplementedError("write your pallas kernel here")


# speedup vs baseline: 5.1535x; 5.1535x over previous
import functools

import jax
import jax.numpy as jnp
from jax import lax
from jax.experimental import pallas as pl
from jax.experimental.pallas import tpu as pltpu
from jax.experimental.pallas import tpu_sc as plsc

N = 4096
D_FEAT = 256
OUT_FEAT = 64
HID = 512
GCN_HID = 512
GCN_OUT = 256
N_NEIGHBORS = 30
SCALE_K = 15
K = 10

F32 = jnp.float32
BM = 256
BMS = 128


def _dot(a, b):
    return jnp.dot(a.astype(jnp.bfloat16), b.astype(jnp.bfloat16),
                   preferred_element_type=F32)


def _dg(a, b, dims):
    return lax.dot_general(a.astype(jnp.bfloat16), b.astype(jnp.bfloat16),
                           (dims, ((), ())), preferred_element_type=F32)


def _dg_hi(a, b, dims):
    return lax.dot_general(a, b, (dims, ((), ())), preferred_element_type=F32,
                           precision=lax.Precision.HIGHEST)


def _gram_body(xo_ref, w1_ref, b1_ref, w2_ref, b2_ref, g_ref):
    i = pl.program_id(0)
    h = jnp.maximum(_dot(xo_ref[...], w1_ref[...]) + b1_ref[...], 0.0)
    yb = _dot(h, w2_ref[...]) + b2_ref[...]
    g = _dg(yb, yb, ((0,), (0,)))

    @pl.when(i == 0)
    def _():
        g_ref[...] = jnp.zeros_like(g_ref)

    g_ref[...] += g


def _gram(x_orth, W1, b1, W2, b2):
    return pl.pallas_call(
        _gram_body,
        grid=(N // BM,),
        in_specs=[
            pl.BlockSpec((BM, D_FEAT), lambda i: (i, 0)),
            pl.BlockSpec((D_FEAT, HID), lambda i: (0, 0)),
            pl.BlockSpec((1, HID), lambda i: (0, 0)),
            pl.BlockSpec((HID, OUT_FEAT), lambda i: (0, 0)),
            pl.BlockSpec((1, OUT_FEAT), lambda i: (0, 0)),
        ],
        out_specs=pl.BlockSpec((OUT_FEAT, OUT_FEAT), lambda i: (0, 0)),
        out_shape=jax.ShapeDtypeStruct((OUT_FEAT, OUT_FEAT), F32),
    )(x_orth, W1, b1, W2, b2)


def _chol_body(g_ref, li_ref):
    f = OUT_FEAT
    rows = lax.broadcasted_iota(jnp.int32, (f, f), 0)
    cols = lax.broadcasted_iota(jnp.int32, (f, f), 1)
    eye = (rows == cols).astype(F32)
    m = g_ref[...] / jnp.float32(N) + 1e-6 * eye

    def chol_step(j, L):
        r = m - _dg_hi(L, L, ((1,), (1,)))
        vcol = jnp.sum(jnp.where(cols == j, r, 0.0), axis=1, keepdims=True)
        dj = jnp.sum(jnp.where((rows == j) & (cols == j), r, 0.0))
        newcol = vcol / jnp.sqrt(dj)
        return jnp.where((cols == j) & (rows >= j), newcol, L)

    L = lax.fori_loop(0, f, chol_step, jnp.zeros((f, f), F32))

    colid = lax.broadcasted_iota(jnp.int32, (1, f), 1)

    def inv_step(j, Li):
        lrow = jnp.sum(jnp.where(rows == j, L, 0.0), axis=0, keepdims=True)
        prod = _dg_hi(lrow, Li, ((1,), (0,)))
        ljj = jnp.sum(jnp.where((rows == j) & (cols == j), L, 0.0))
        ej = (colid == j).astype(F32)
        newrow = (ej - prod) / ljj
        return jnp.where(rows == j, newrow, Li)

    li_ref[...] = lax.fori_loop(0, f, inv_step, jnp.zeros((f, f), F32))


def _chol_inv(G):
    return pl.pallas_call(
        _chol_body,
        out_shape=jax.ShapeDtypeStruct((OUT_FEAT, OUT_FEAT), F32),
    )(G)


def _mlp_body(x_ref, w1_ref, b1_ref, w2_ref, b2_ref, li_ref, wg1_ref,
              os_ref, y_ref, xg_ref):
    xb = x_ref[...]
    h = jnp.maximum(_dot(xb, w1_ref[...]) + b1_ref[...], 0.0)
    s = _dot(h, w2_ref[...]) + b2_ref[...]
    o = _dg(s, li_ref[...], ((1,), (1,)))
    os_ref[...] = jnp.concatenate([o, s], axis=1)
    y_ref[...] = 64.0 * o
    xg_ref[...] = _dot(xb, wg1_ref[...])


def _mlp_all(x, W1, b1, W2, b2, Linv, Wg1):
    return pl.pallas_call(
        _mlp_body,
        grid=(N // BM,),
        in_specs=[
            pl.BlockSpec((BM, D_FEAT), lambda i: (i, 0)),
            pl.BlockSpec((D_FEAT, HID), lambda i: (0, 0)),
            pl.BlockSpec((1, HID), lambda i: (0, 0)),
            pl.BlockSpec((HID, OUT_FEAT), lambda i: (0, 0)),
            pl.BlockSpec((1, OUT_FEAT), lambda i: (0, 0)),
            pl.BlockSpec((OUT_FEAT, OUT_FEAT), lambda i: (0, 0)),
            pl.BlockSpec((D_FEAT, GCN_HID), lambda i: (0, 0)),
        ],
        out_specs=[
            pl.BlockSpec((BM, 2 * OUT_FEAT), lambda i: (i, 0)),
            pl.BlockSpec((BM, OUT_FEAT), lambda i: (i, 0)),
            pl.BlockSpec((BM, GCN_HID), lambda i: (i, 0)),
        ],
        out_shape=[
            jax.ShapeDtypeStruct((N, 2 * OUT_FEAT), F32),
            jax.ShapeDtypeStruct((N, OUT_FEAT), F32),
            jax.ShapeDtypeStruct((N, GCN_HID), F32),
        ],
    )(x, W1, b1, W2, b2, Linv, Wg1)


def _kth_extract(mat, k, kind):
    rows = mat.shape[0]
    sentinel = jnp.float32(3.0e38) if kind == "min" else jnp.float32(-3.0e38)

    def step(_, carry):
        work, t, cc = carry
        if kind == "min":
            m = jnp.min(work, axis=1, keepdims=True)
        else:
            m = jnp.max(work, axis=1, keepdims=True)
        eq = work == m
        cnt = jnp.sum(eq.astype(F32), axis=1, keepdims=True)
        cc2 = cc + cnt
        kf = jnp.float32(k)
        sel = (cc < kf) & (cc2 >= kf)
        t = jnp.where(sel, m, t)
        work = jnp.where(eq, sentinel, work)
        return work, t, cc2

    _, t, _ = lax.fori_loop(
        0, k, step,
        (mat, jnp.zeros((rows, 1), F32), jnp.zeros((rows, 1), F32)))
    return t


def _d2_panel(xb, xf):
    sqb = jnp.sum(xb * xb, axis=1, keepdims=True)
    sqf = jnp.sum(xf * xf, axis=1)
    panel = _dg(xb, xf, ((1,), (1,)))
    return jnp.maximum(sqb + sqf[None, :] - 2.0 * panel, 0.0)


def _col_to_row(col):
    n = col.shape[0]
    r = lax.broadcasted_iota(jnp.int32, (n, n), 0)
    c = lax.broadcasted_iota(jnp.int32, (n, n), 1)
    eye = (r == c).astype(F32)
    return _dg_hi(col, eye, ((0,), (0,)))


def _row_to_col(row):
    n = row.shape[1]
    r = lax.broadcasted_iota(jnp.int32, (n, n), 0)
    c = lax.broadcasted_iota(jnp.int32, (n, n), 1)
    eye = (r == c).astype(F32)
    return _dg_hi(eye, row, ((0,), (1,)))


def _scale_body(x_ref, xf_ref, sc_ref, sr_ref):
    d2 = _d2_panel(x_ref[...], xf_ref[...])
    t = _kth_extract(d2, SCALE_K + 1, "min")
    s = jnp.sqrt(t + 1e-12)
    sc_ref[...] = s
    sr_ref[...] = _col_to_row(s)


def _scale_kernel(x):
    return pl.pallas_call(
        _scale_body,
        grid=(N // BMS,),
        in_specs=[
            pl.BlockSpec((BMS, D_FEAT), lambda i: (i, 0)),
            pl.BlockSpec((N, D_FEAT), lambda i: (0, 0)),
        ],
        out_specs=[
            pl.BlockSpec((BMS, 1), lambda i: (i, 0)),
            pl.BlockSpec((1, BMS), lambda i: (0, i)),
        ],
        out_shape=[
            jax.ShapeDtypeStruct((N, 1), F32),
            jax.ShapeDtypeStruct((1, N), F32),
        ],
    )(x, x)


def _wm_body(x_ref, xf_ref, sc_ref, sr_ref, wm_ref, rs_ref, cs_ref):
    i = pl.program_id(0)
    d2 = _d2_panel(x_ref[...], xf_ref[...])
    wm = jnp.exp(-d2 / (sc_ref[...] * sr_ref[...] + 1e-8))
    thr = _kth_extract(wm, N_NEIGHBORS + 1, "max")
    wmm = wm * (wm >= thr).astype(F32)
    wm_ref[...] = wmm
    rs = jnp.sum(wmm, axis=1, keepdims=True)
    rs_ref[...] = _col_to_row(rs)

    @pl.when(i == 0)
    def _():
        cs_ref[...] = jnp.zeros_like(cs_ref)

    cs_ref[...] += jnp.sum(wmm, axis=0, keepdims=True)


def _wm_kernel(x, scale_col, scale_row):
    return pl.pallas_call(
        _wm_body,
        grid=(N // BMS,),
        in_specs=[
            pl.BlockSpec((BMS, D_FEAT), lambda i: (i, 0)),
            pl.BlockSpec((N, D_FEAT), lambda i: (0, 0)),
            pl.BlockSpec((BMS, 1), lambda i: (i, 0)),
            pl.BlockSpec((1, N), lambda i: (0, 0)),
        ],
        out_specs=[
            pl.BlockSpec((BMS, N), lambda i: (i, 0)),
            pl.BlockSpec((1, BMS), lambda i: (0, i)),
            pl.BlockSpec((1, N), lambda i: (0, 0)),
        ],
        out_shape=[
            jax.ShapeDtypeStruct((N, N), F32),
            jax.ShapeDtypeStruct((1, N), F32),
            jax.ShapeDtypeStruct((1, N), F32),
        ],
    )(x, x, scale_col, scale_row)


def _prep_body(rs_ref, cs_ref, xg_ref, dinv_ref, c_ref):
    i = pl.program_id(0)
    drow = 0.5 * (rs_ref[...] + cs_ref[...])
    dinv = 1.0 / jnp.sqrt(drow + 1e-8)
    dinv_ref[...] = dinv
    c_ref[...] = xg_ref[...] * _row_to_col(dinv)


def _prep_kernel(rowsum_row, colsum_row, Xg1):
    return pl.pallas_call(
        _prep_body,
        grid=(N // BM,),
        in_specs=[
            pl.BlockSpec((1, BM), lambda i: (0, i)),
            pl.BlockSpec((1, BM), lambda i: (0, i)),
            pl.BlockSpec((BM, GCN_HID), lambda i: (i, 0)),
        ],
        out_specs=[
            pl.BlockSpec((1, BM), lambda i: (0, i)),
            pl.BlockSpec((BM, GCN_HID), lambda i: (i, 0)),
        ],
        out_shape=[
            jax.ShapeDtypeStruct((1, N), F32),
            jax.ShapeDtypeStruct((N, GCN_HID), F32),
        ],
    )(rowsum_row, colsum_row, Xg1)


def _gcn1_body(wr_ref, wc_ref, c_ref, dv_ref, wg2_ref, c2_ref):
    dcol = _row_to_col(dv_ref[...])
    t1 = _dot(wr_ref[...], c_ref[...])
    t2 = _dg(wc_ref[...], c_ref[...], ((0,), (0,)))
    h = jnp.maximum(0.5 * dcol * (t1 + t2), 0.0)
    c2_ref[...] = dcol * _dot(h, wg2_ref[...])


def _gcn1_kernel(Wmm, C, dinv_row, Wg2):
    return pl.pallas_call(
        _gcn1_body,
        grid=(N // BM,),
        in_specs=[
            pl.BlockSpec((BM, N), lambda i: (i, 0)),
            pl.BlockSpec((N, BM), lambda i: (0, i)),
            pl.BlockSpec((N, GCN_HID), lambda i: (0, 0)),
            pl.BlockSpec((1, BM), lambda i: (0, i)),
            pl.BlockSpec((GCN_HID, GCN_OUT), lambda i: (0, 0)),
        ],
        out_specs=pl.BlockSpec((BM, GCN_OUT), lambda i: (i, 0)),
        out_shape=jax.ShapeDtypeStruct((N, GCN_OUT), F32),
    )(Wmm, Wmm, C, dinv_row, Wg2)


def _gcn2_body(wr_ref, wc_ref, c2_ref, dv_ref, out_ref):
    dcol = _row_to_col(dv_ref[...])
    t1 = _dot(wr_ref[...], c2_ref[...])
    t2 = _dg(wc_ref[...], c2_ref[...], ((0,), (0,)))
    out_ref[...] = 0.5 * dcol * (t1 + t2)


def _gcn2_kernel(Wmm, C2, dinv_row):
    return pl.pallas_call(
        _gcn2_body,
        grid=(N // BM,),
        in_specs=[
            pl.BlockSpec((BM, N), lambda i: (i, 0)),
            pl.BlockSpec((N, BM), lambda i: (0, i)),
            pl.BlockSpec((N, GCN_OUT), lambda i: (0, 0)),
            pl.BlockSpec((1, BM), lambda i: (0, i)),
        ],
        out_specs=pl.BlockSpec((BM, GCN_OUT), lambda i: (i, 0)),
        out_shape=jax.ShapeDtypeStruct((N, GCN_OUT), F32),
    )(Wmm, Wmm, C2, dinv_row)


ROWS_PW = N // 32
CH = 8
NCHUNK = ROWS_PW // CH


def _sc_sqrt(x):
    i = plsc.bitcast(x, jnp.int32)
    r = plsc.bitcast(jnp.int32(0x5F3759DF) - (i >> 1), F32)
    for _ in range(3):
        r = r * (1.5 - 0.5 * x * r * r)
    s = x * r
    return 0.5 * (s + x / s)


def _sc_knn_body(os_hbm, idx_hbm, idxf_hbm, bet_hbm, alp_hbm,
                 a_hbm, eh_hbm,
                 idx_v, idx_f, selfo, g_os, stripe, ebuf, ib,
                 scal, sem_g, sem_s):
    wid = lax.axis_index("s") * 2 + lax.axis_index("c")
    base = wid * ROWS_PW

    pltpu.sync_copy(idx_hbm.at[pl.ds(base, ROWS_PW)], idx_v)
    pltpu.sync_copy(idxf_hbm.at[pl.ds(base * (K + 6), ROWS_PW * (K + 6))],
                    idx_f)
    pltpu.sync_copy(bet_hbm, scal)
    betav = scal[...]
    pltpu.sync_copy(alp_hbm, scal)
    alpav = scal[...]

    ii = lax.broadcasted_iota(jnp.int32, (16,), 0)
    valid = (ii >= 1) & (ii <= K)
    zeros16 = jnp.zeros((16,), F32)
    jf = (ii + 1).astype(F32)

    def zstep(t, _):
        stripe[t // (N // 16), pl.ds((t % (N // 16)) * 16, 16)] = zeros16
        return 0
    lax.fori_loop(0, CH * (N // 16), zstep, 0)

    def chunk_step(c, _):
        row0 = base + c * CH
        gcp = pltpu.async_copy(os_hbm.at[idx_f.at[pl.ds(c * CH * 16, CH * 16)]],
                               g_os, sem_g)
        pltpu.sync_copy(os_hbm.at[pl.ds(row0, CH)], selfo)
        gcp.wait()

        def row_step(r, _):
            i0 = c * CH + r
            idxvec = idx_v[i0]
            d2 = zeros16
            for j in range(1, K + 1):
                acc = zeros16
                for q in range(4):
                    t = g_os[r * 16 + j, pl.ds(q * 16, 16)] - selfo[r, pl.ds(q * 16, 16)]
                    acc = acc + t * t
                d2 = jnp.where(ii == j, jnp.sum(acc), d2)
            dxi = _sc_sqrt(d2 + 1e-8)
            dfi = _sc_sqrt(4096.0 * d2 + 1e-12)
            ad = -(dxi + betav * dfi) / (2.0 * alpav)
            adm = jnp.where(valid, ad, -3.0e38)
            u, _unused = plsc.sort_key_val(adm, adm, descending=True)
            css = plsc.cumsum(u)
            cond = ((u + (1.0 - css) / jf) > 0) & (ii < K)
            rho = plsc.all_reduce_population_count(cond)
            rhof = rho.astype(F32)
            cssrho = jnp.sum(jnp.where(ii == rho - 1, css, 0.0))
            theta = (cssrho - 1.0) / rhof
            vals = jnp.where(valid, jnp.maximum(ad - theta, 0.0), 0.0)
            ib[...] = idxvec
            dup = ii < 0
            for s in range(1, K):
                sh = plsc.load_gather(ib, [jnp.minimum(ii + s, 15)])
                dup = dup | ((idxvec == sh) & (ii >= 1) & (ii + s <= K))
            wvals = jnp.where(valid & (~dup), vals, 0.0)
            rsplat = jnp.broadcast_to(r, (16,)).astype(jnp.int32)
            plsc.store_scatter(stripe, [rsplat, idxvec], vals, mask=valid)
            ib[...] = plsc.bitcast(wvals, jnp.int32)
            accs = [zeros16] * 4
            for j in range(1, K + 1):
                wj = plsc.bitcast(
                    plsc.load_gather(ib, [jnp.full((16,), j, jnp.int32)]), F32)
                for q in range(4):
                    accs[q] = accs[q] + wj * g_os[r * 16 + j, pl.ds(64 + q * 16, 16)]
            for q in range(4):
                ebuf[i0, pl.ds(q * 16, 16)] = accs[q]
            return 0

        lax.fori_loop(0, CH, row_step, 0)
        pltpu.sync_copy(stripe, a_hbm.at[pl.ds(row0, CH)])

        def rz_step(r, _):
            idxvec = idx_v[c * CH + r]
            rsplat = jnp.broadcast_to(r, (16,)).astype(jnp.int32)
            plsc.store_scatter(stripe, [rsplat, idxvec], zeros16, mask=valid)
            return 0
        lax.fori_loop(0, CH, rz_step, 0)
        return 0

    lax.fori_loop(0, NCHUNK, chunk_step, 0)
    pltpu.sync_copy(ebuf, eh_hbm.at[pl.ds(base, ROWS_PW)])


def _sc_knn(OS, idx, beta16, alpha16):
    mesh = plsc.VectorSubcoreMesh(core_axis_name="c", subcore_axis_name="s")
    f = functools.partial(
        pl.kernel,
        out_type=[
            jax.ShapeDtypeStruct((N, N), F32),
            jax.ShapeDtypeStruct((N, OUT_FEAT), F32),
        ],
        mesh=mesh,
        compiler_params=pltpu.CompilerParams(needs_layout_passes=False),
        scratch_types=[
            pltpu.VMEM((ROWS_PW, K + 6), jnp.int32),
            pltpu.VMEM((ROWS_PW * (K + 6),), jnp.int32),
            pltpu.VMEM((CH, 2 * OUT_FEAT), F32),
            pltpu.VMEM((CH * 16, 2 * OUT_FEAT), F32),
            pltpu.VMEM((CH, N), F32),
            pltpu.VMEM((ROWS_PW, OUT_FEAT), F32),
            pltpu.VMEM((16,), jnp.int32),
            pltpu.VMEM((16,), F32),
            pltpu.SemaphoreType.DMA,
            pltpu.SemaphoreType.DMA,
        ],
    )(_sc_knn_body)
    idx_flat = idx.reshape(N * (K + 6))
    return f(OS, idx, idx_flat, beta16, alpha16)


def kernel(x, x_orth, beta, alpha, idx, W1, b1, W2, b2, Wg1, Wg2):
    b1r = b1.reshape(1, HID)
    b2r = b2.reshape(1, OUT_FEAT)
    idx = idx.astype(jnp.int32)
    beta16 = jnp.full((16,), beta, F32)
    alpha16 = jnp.full((16,), alpha, F32)

    G = _gram(x_orth, W1, b1r, W2, b2r)
    Linv = _chol_inv(G)
    OS, Y, Xg1 = _mlp_all(x, W1, b1r, W2, b2r, Linv, Wg1)

    scale_col, scale_row = _scale_kernel(x)
    Wmm, rowsum_row, colsum_row = _wm_kernel(x, scale_col, scale_row)
    dinv_row, C = _prep_kernel(rowsum_row, colsum_row, Xg1)
    C2 = _gcn1_kernel(Wmm, C, dinv_row, Wg2)
    embs_graph = _gcn2_kernel(Wmm, C2, dinv_row)

    A, embs_hom = _sc_knn(OS, idx, beta16, alpha16)
    return (embs_hom, embs_graph, A, Y)


# speedup vs baseline: 8.5640x; 1.6618x over previous
import functools

import jax
import jax.numpy as jnp
from jax import lax
from jax.experimental import pallas as pl
from jax.experimental.pallas import tpu as pltpu
from jax.experimental.pallas import tpu_sc as plsc

N = 4096
D_FEAT = 256
OUT_FEAT = 64
HID = 512
GCN_HID = 512
GCN_OUT = 256
N_NEIGHBORS = 30
SCALE_K = 15
K = 10

F32 = jnp.float32
BM = 256
BMS = 128


def _dot(a, b):
    return jnp.dot(a.astype(jnp.bfloat16), b.astype(jnp.bfloat16),
                   preferred_element_type=F32)


def _dg(a, b, dims):
    return lax.dot_general(a.astype(jnp.bfloat16), b.astype(jnp.bfloat16),
                           (dims, ((), ())), preferred_element_type=F32)


def _dg_hi(a, b, dims):
    return lax.dot_general(a, b, (dims, ((), ())), preferred_element_type=F32,
                           precision=lax.Precision.HIGHEST)


def _gram_body(xo_ref, w1_ref, b1_ref, w2_ref, b2_ref, g_ref):
    i = pl.program_id(0)
    h = jnp.maximum(_dot(xo_ref[...], w1_ref[...]) + b1_ref[...], 0.0)
    yb = _dot(h, w2_ref[...]) + b2_ref[...]
    g = _dg(yb, yb, ((0,), (0,)))

    @pl.when(i == 0)
    def _():
        g_ref[...] = jnp.zeros_like(g_ref)

    g_ref[...] += g


def _gram(x_orth, W1, b1, W2, b2):
    return pl.pallas_call(
        _gram_body,
        grid=(N // BM,),
        in_specs=[
            pl.BlockSpec((BM, D_FEAT), lambda i: (i, 0)),
            pl.BlockSpec((D_FEAT, HID), lambda i: (0, 0)),
            pl.BlockSpec((1, HID), lambda i: (0, 0)),
            pl.BlockSpec((HID, OUT_FEAT), lambda i: (0, 0)),
            pl.BlockSpec((1, OUT_FEAT), lambda i: (0, 0)),
        ],
        out_specs=pl.BlockSpec((OUT_FEAT, OUT_FEAT), lambda i: (0, 0)),
        out_shape=jax.ShapeDtypeStruct((OUT_FEAT, OUT_FEAT), F32),
    )(x_orth, W1, b1, W2, b2)


def _chol_body(g_ref, li_ref):
    f = OUT_FEAT
    rows = lax.broadcasted_iota(jnp.int32, (f, f), 0)
    cols = lax.broadcasted_iota(jnp.int32, (f, f), 1)
    eye = (rows == cols).astype(F32)
    m = g_ref[...] / jnp.float32(N) + 1e-6 * eye

    def chol_step(j, L):
        r = m - _dg_hi(L, L, ((1,), (1,)))
        vcol = jnp.sum(jnp.where(cols == j, r, 0.0), axis=1, keepdims=True)
        dj = jnp.sum(jnp.where((rows == j) & (cols == j), r, 0.0))
        newcol = vcol / jnp.sqrt(dj)
        return jnp.where((cols == j) & (rows >= j), newcol, L)

    L = lax.fori_loop(0, f, chol_step, jnp.zeros((f, f), F32))

    colid = lax.broadcasted_iota(jnp.int32, (1, f), 1)

    def inv_step(j, Li):
        lrow = jnp.sum(jnp.where(rows == j, L, 0.0), axis=0, keepdims=True)
        prod = _dg_hi(lrow, Li, ((1,), (0,)))
        ljj = jnp.sum(jnp.where((rows == j) & (cols == j), L, 0.0))
        ej = (colid == j).astype(F32)
        newrow = (ej - prod) / ljj
        return jnp.where(rows == j, newrow, Li)

    li_ref[...] = lax.fori_loop(0, f, inv_step, jnp.zeros((f, f), F32))


def _chol_inv(G):
    return pl.pallas_call(
        _chol_body,
        out_shape=jax.ShapeDtypeStruct((OUT_FEAT, OUT_FEAT), F32),
    )(G)


def _mlp_body(x_ref, w1_ref, b1_ref, w2_ref, b2_ref, li_ref, wg1_ref,
              os_ref, y_ref, xg_ref):
    xb = x_ref[...]
    h = jnp.maximum(_dot(xb, w1_ref[...]) + b1_ref[...], 0.0)
    s = _dot(h, w2_ref[...]) + b2_ref[...]
    o = _dg(s, li_ref[...], ((1,), (1,)))
    os_ref[...] = jnp.concatenate([o, s], axis=1)
    y_ref[...] = 64.0 * o
    xg_ref[...] = _dot(xb, wg1_ref[...])


def _mlp_all(x, W1, b1, W2, b2, Linv, Wg1):
    return pl.pallas_call(
        _mlp_body,
        grid=(N // BM,),
        in_specs=[
            pl.BlockSpec((BM, D_FEAT), lambda i: (i, 0)),
            pl.BlockSpec((D_FEAT, HID), lambda i: (0, 0)),
            pl.BlockSpec((1, HID), lambda i: (0, 0)),
            pl.BlockSpec((HID, OUT_FEAT), lambda i: (0, 0)),
            pl.BlockSpec((1, OUT_FEAT), lambda i: (0, 0)),
            pl.BlockSpec((OUT_FEAT, OUT_FEAT), lambda i: (0, 0)),
            pl.BlockSpec((D_FEAT, GCN_HID), lambda i: (0, 0)),
        ],
        out_specs=[
            pl.BlockSpec((BM, 2 * OUT_FEAT), lambda i: (i, 0)),
            pl.BlockSpec((BM, OUT_FEAT), lambda i: (i, 0)),
            pl.BlockSpec((BM, GCN_HID), lambda i: (i, 0)),
        ],
        out_shape=[
            jax.ShapeDtypeStruct((N, 2 * OUT_FEAT), F32),
            jax.ShapeDtypeStruct((N, OUT_FEAT), F32),
            jax.ShapeDtypeStruct((N, GCN_HID), F32),
        ],
    )(x, W1, b1, W2, b2, Linv, Wg1)


def _kth_extract(mat, k, kind):
    rows = mat.shape[0]
    bits = lax.bitcast_convert_type(mat, jnp.int32)
    kf = jnp.float32(k)
    if kind == "min":
        lo = jnp.full((rows, 1), -1, jnp.int32)
        hi = jnp.full((rows, 1), 0x7F7FFFFF, jnp.int32)

        def step(_, carry):
            lo, hi = carry
            mid = lo + ((hi - lo) >> 1)
            cnt = jnp.sum((bits <= mid).astype(F32), axis=1, keepdims=True)
            pred = cnt >= kf
            return jnp.where(pred, lo, mid), jnp.where(pred, mid, hi)

        lo, hi = lax.fori_loop(0, 31, step, (lo, hi))
        ans = hi
    else:
        lo = jnp.zeros((rows, 1), jnp.int32)
        hi = jnp.full((rows, 1), 0x7F800000, jnp.int32)

        def step(_, carry):
            lo, hi = carry
            mid = lo + ((hi - lo) >> 1)
            cnt = jnp.sum((bits >= mid).astype(F32), axis=1, keepdims=True)
            pred = cnt >= kf
            return jnp.where(pred, mid, lo), jnp.where(pred, hi, mid)

        lo, hi = lax.fori_loop(0, 31, step, (lo, hi))
        ans = lo
    return lax.bitcast_convert_type(ans, F32)


def _d2_panel(xb, xf):
    sqb = jnp.sum(xb * xb, axis=1, keepdims=True)
    sqf = jnp.sum(xf * xf, axis=1)
    panel = _dg(xb, xf, ((1,), (1,)))
    return jnp.maximum(sqb + sqf[None, :] - 2.0 * panel, 0.0)


def _col_to_row(col):
    n = col.shape[0]
    r = lax.broadcasted_iota(jnp.int32, (n, n), 0)
    c = lax.broadcasted_iota(jnp.int32, (n, n), 1)
    eye = (r == c).astype(F32)
    return _dg_hi(col, eye, ((0,), (0,)))


def _row_to_col(row):
    n = row.shape[1]
    r = lax.broadcasted_iota(jnp.int32, (n, n), 0)
    c = lax.broadcasted_iota(jnp.int32, (n, n), 1)
    eye = (r == c).astype(F32)
    return _dg_hi(eye, row, ((0,), (1,)))


def _scale_body(x_ref, xf_ref, sc_ref, sr_ref):
    d2 = _d2_panel(x_ref[...], xf_ref[...])
    t = _kth_extract(d2, SCALE_K + 1, "min")
    s = jnp.sqrt(t + 1e-12)
    sc_ref[...] = s
    sr_ref[...] = _col_to_row(s)


def _scale_kernel(x):
    return pl.pallas_call(
        _scale_body,
        grid=(N // BMS,),
        in_specs=[
            pl.BlockSpec((BMS, D_FEAT), lambda i: (i, 0)),
            pl.BlockSpec((N, D_FEAT), lambda i: (0, 0)),
        ],
        out_specs=[
            pl.BlockSpec((BMS, 1), lambda i: (i, 0)),
            pl.BlockSpec((1, BMS), lambda i: (0, i)),
        ],
        out_shape=[
            jax.ShapeDtypeStruct((N, 1), F32),
            jax.ShapeDtypeStruct((1, N), F32),
        ],
    )(x, x)


def _wm_body(x_ref, xf_ref, sc_ref, sr_ref, wm_ref, rs_ref, cs_ref):
    i = pl.program_id(0)
    d2 = _d2_panel(x_ref[...], xf_ref[...])
    wm = jnp.exp(-d2 / (sc_ref[...] * sr_ref[...] + 1e-8))
    thr = _kth_extract(wm, N_NEIGHBORS + 1, "max")
    wmm = wm * (wm >= thr).astype(F32)
    wm_ref[...] = wmm
    rs = jnp.sum(wmm, axis=1, keepdims=True)
    rs_ref[...] = _col_to_row(rs)

    @pl.when(i == 0)
    def _():
        cs_ref[...] = jnp.zeros_like(cs_ref)

    cs_ref[...] += jnp.sum(wmm, axis=0, keepdims=True)


def _wm_kernel(x, scale_col, scale_row):
    return pl.pallas_call(
        _wm_body,
        grid=(N // BMS,),
        in_specs=[
            pl.BlockSpec((BMS, D_FEAT), lambda i: (i, 0)),
            pl.BlockSpec((N, D_FEAT), lambda i: (0, 0)),
            pl.BlockSpec((BMS, 1), lambda i: (i, 0)),
            pl.BlockSpec((1, N), lambda i: (0, 0)),
        ],
        out_specs=[
            pl.BlockSpec((BMS, N), lambda i: (i, 0)),
            pl.BlockSpec((1, BMS), lambda i: (0, i)),
            pl.BlockSpec((1, N), lambda i: (0, 0)),
        ],
        out_shape=[
            jax.ShapeDtypeStruct((N, N), F32),
            jax.ShapeDtypeStruct((1, N), F32),
            jax.ShapeDtypeStruct((1, N), F32),
        ],
    )(x, x, scale_col, scale_row)


def _prep_body(rs_ref, cs_ref, xg_ref, dinv_ref, c_ref):
    i = pl.program_id(0)
    drow = 0.5 * (rs_ref[...] + cs_ref[...])
    dinv = 1.0 / jnp.sqrt(drow + 1e-8)
    dinv_ref[...] = dinv
    c_ref[...] = xg_ref[...] * _row_to_col(dinv)


def _prep_kernel(rowsum_row, colsum_row, Xg1):
    return pl.pallas_call(
        _prep_body,
        grid=(N // BM,),
        in_specs=[
            pl.BlockSpec((1, BM), lambda i: (0, i)),
            pl.BlockSpec((1, BM), lambda i: (0, i)),
            pl.BlockSpec((BM, GCN_HID), lambda i: (i, 0)),
        ],
        out_specs=[
            pl.BlockSpec((1, BM), lambda i: (0, i)),
            pl.BlockSpec((BM, GCN_HID), lambda i: (i, 0)),
        ],
        out_shape=[
            jax.ShapeDtypeStruct((1, N), F32),
            jax.ShapeDtypeStruct((N, GCN_HID), F32),
        ],
    )(rowsum_row, colsum_row, Xg1)


def _gcn1_body(wr_ref, wc_ref, c_ref, dv_ref, wg2_ref, c2_ref):
    dcol = _row_to_col(dv_ref[...])
    t1 = _dot(wr_ref[...], c_ref[...])
    t2 = _dg(wc_ref[...], c_ref[...], ((0,), (0,)))
    h = jnp.maximum(0.5 * dcol * (t1 + t2), 0.0)
    c2_ref[...] = dcol * _dot(h, wg2_ref[...])


def _gcn1_kernel(Wmm, C, dinv_row, Wg2):
    return pl.pallas_call(
        _gcn1_body,
        grid=(N // BM,),
        in_specs=[
            pl.BlockSpec((BM, N), lambda i: (i, 0)),
            pl.BlockSpec((N, BM), lambda i: (0, i)),
            pl.BlockSpec((N, GCN_HID), lambda i: (0, 0)),
            pl.BlockSpec((1, BM), lambda i: (0, i)),
            pl.BlockSpec((GCN_HID, GCN_OUT), lambda i: (0, 0)),
        ],
        out_specs=pl.BlockSpec((BM, GCN_OUT), lambda i: (i, 0)),
        out_shape=jax.ShapeDtypeStruct((N, GCN_OUT), F32),
    )(Wmm, Wmm, C, dinv_row, Wg2)


def _gcn2_body(wr_ref, wc_ref, c2_ref, dv_ref, out_ref):
    dcol = _row_to_col(dv_ref[...])
    t1 = _dot(wr_ref[...], c2_ref[...])
    t2 = _dg(wc_ref[...], c2_ref[...], ((0,), (0,)))
    out_ref[...] = 0.5 * dcol * (t1 + t2)


def _gcn2_kernel(Wmm, C2, dinv_row):
    return pl.pallas_call(
        _gcn2_body,
        grid=(N // BM,),
        in_specs=[
            pl.BlockSpec((BM, N), lambda i: (i, 0)),
            pl.BlockSpec((N, BM), lambda i: (0, i)),
            pl.BlockSpec((N, GCN_OUT), lambda i: (0, 0)),
            pl.BlockSpec((1, BM), lambda i: (0, i)),
        ],
        out_specs=pl.BlockSpec((BM, GCN_OUT), lambda i: (i, 0)),
        out_shape=jax.ShapeDtypeStruct((N, GCN_OUT), F32),
    )(Wmm, Wmm, C2, dinv_row)


ROWS_PW = N // 32
CH = 8
NCHUNK = ROWS_PW // CH


def _sc_sqrt(x):
    i = plsc.bitcast(x, jnp.int32)
    r = plsc.bitcast(jnp.int32(0x5F3759DF) - (i >> 1), F32)
    for _ in range(3):
        r = r * (1.5 - 0.5 * x * r * r)
    s = x * r
    return 0.5 * (s + x / s)


def _sc_knn_body(os_hbm, idx_hbm, idxf_hbm, bet_hbm, alp_hbm,
                 a_hbm, eh_hbm,
                 idx_v, idx_f, selfo, g_os, stripe, ebuf, ib,
                 scal, sem_g, sem_s):
    wid = lax.axis_index("s") * 2 + lax.axis_index("c")
    base = wid * ROWS_PW

    pltpu.sync_copy(idx_hbm.at[pl.ds(base, ROWS_PW)], idx_v)
    pltpu.sync_copy(idxf_hbm.at[pl.ds(base * (K + 6), ROWS_PW * (K + 6))],
                    idx_f)
    pltpu.sync_copy(bet_hbm, scal)
    betav = scal[...]
    pltpu.sync_copy(alp_hbm, scal)
    alpav = scal[...]

    ii = lax.broadcasted_iota(jnp.int32, (16,), 0)
    valid = (ii >= 1) & (ii <= K)
    zeros16 = jnp.zeros((16,), F32)
    jf = (ii + 1).astype(F32)

    def zstep(t, _):
        stripe[t // (N // 16), pl.ds((t % (N // 16)) * 16, 16)] = zeros16
        return 0
    lax.fori_loop(0, CH * (N // 16), zstep, 0)

    def chunk_step(c, _):
        row0 = base + c * CH
        gcp = pltpu.async_copy(os_hbm.at[idx_f.at[pl.ds(c * CH * 16, CH * 16)]],
                               g_os, sem_g)
        pltpu.sync_copy(os_hbm.at[pl.ds(row0, CH)], selfo)
        gcp.wait()

        def row_step(r, _):
            i0 = c * CH + r
            idxvec = idx_v[i0]
            d2 = zeros16
            for j in range(1, K + 1):
                acc = zeros16
                for q in range(4):
                    t = g_os[r * 16 + j, pl.ds(q * 16, 16)] - selfo[r, pl.ds(q * 16, 16)]
                    acc = acc + t * t
                d2 = jnp.where(ii == j, jnp.sum(acc), d2)
            dxi = _sc_sqrt(d2 + 1e-8)
            dfi = _sc_sqrt(4096.0 * d2 + 1e-12)
            ad = -(dxi + betav * dfi) / (2.0 * alpav)
            adm = jnp.where(valid, ad, -3.0e38)
            u, _unused = plsc.sort_key_val(adm, adm, descending=True)
            css = plsc.cumsum(u)
            cond = ((u + (1.0 - css) / jf) > 0) & (ii < K)
            rho = plsc.all_reduce_population_count(cond)
            rhof = rho.astype(F32)
            cssrho = jnp.sum(jnp.where(ii == rho - 1, css, 0.0))
            theta = (cssrho - 1.0) / rhof
            vals = jnp.where(valid, jnp.maximum(ad - theta, 0.0), 0.0)
            ib[...] = idxvec
            dup = ii < 0
            for s in range(1, K):
                sh = plsc.load_gather(ib, [jnp.minimum(ii + s, 15)])
                dup = dup | ((idxvec == sh) & (ii >= 1) & (ii + s <= K))
            wvals = jnp.where(valid & (~dup), vals, 0.0)
            rsplat = jnp.broadcast_to(r, (16,)).astype(jnp.int32)
            plsc.store_scatter(stripe, [rsplat, idxvec], vals, mask=valid)
            ib[...] = plsc.bitcast(wvals, jnp.int32)
            accs = [zeros16] * 4
            for j in range(1, K + 1):
                wj = plsc.bitcast(
                    plsc.load_gather(ib, [jnp.full((16,), j, jnp.int32)]), F32)
                for q in range(4):
                    accs[q] = accs[q] + wj * g_os[r * 16 + j, pl.ds(64 + q * 16, 16)]
            for q in range(4):
                ebuf[i0, pl.ds(q * 16, 16)] = accs[q]
            return 0

        lax.fori_loop(0, CH, row_step, 0)
        pltpu.sync_copy(stripe, a_hbm.at[pl.ds(row0, CH)])

        def rz_step(r, _):
            idxvec = idx_v[c * CH + r]
            rsplat = jnp.broadcast_to(r, (16,)).astype(jnp.int32)
            plsc.store_scatter(stripe, [rsplat, idxvec], zeros16, mask=valid)
            return 0
        lax.fori_loop(0, CH, rz_step, 0)
        return 0

    lax.fori_loop(0, NCHUNK, chunk_step, 0)
    pltpu.sync_copy(ebuf, eh_hbm.at[pl.ds(base, ROWS_PW)])


def _sc_knn(OS, idx, beta16, alpha16):
    mesh = plsc.VectorSubcoreMesh(core_axis_name="c", subcore_axis_name="s")
    f = functools.partial(
        pl.kernel,
        out_type=[
            jax.ShapeDtypeStruct((N, N), F32),
            jax.ShapeDtypeStruct((N, OUT_FEAT), F32),
        ],
        mesh=mesh,
        compiler_params=pltpu.CompilerParams(needs_layout_passes=False),
        scratch_types=[
            pltpu.VMEM((ROWS_PW, K + 6), jnp.int32),
            pltpu.VMEM((ROWS_PW * (K + 6),), jnp.int32),
            pltpu.VMEM((CH, 2 * OUT_FEAT), F32),
            pltpu.VMEM((CH * 16, 2 * OUT_FEAT), F32),
            pltpu.VMEM((CH, N), F32),
            pltpu.VMEM((ROWS_PW, OUT_FEAT), F32),
            pltpu.VMEM((16,), jnp.int32),
            pltpu.VMEM((16,), F32),
            pltpu.SemaphoreType.DMA,
            pltpu.SemaphoreType.DMA,
        ],
    )(_sc_knn_body)
    idx_flat = idx.reshape(N * (K + 6))
    return f(OS, idx, idx_flat, beta16, alpha16)


def kernel(x, x_orth, beta, alpha, idx, W1, b1, W2, b2, Wg1, Wg2):
    b1r = b1.reshape(1, HID)
    b2r = b2.reshape(1, OUT_FEAT)
    idx = idx.astype(jnp.int32)
    beta16 = jnp.full((16,), beta, F32)
    alpha16 = jnp.full((16,), alpha, F32)

    G = _gram(x_orth, W1, b1r, W2, b2r)
    Linv = _chol_inv(G)
    OS, Y, Xg1 = _mlp_all(x, W1, b1r, W2, b2r, Linv, Wg1)

    scale_col, scale_row = _scale_kernel(x)
    Wmm, rowsum_row, colsum_row = _wm_kernel(x, scale_col, scale_row)
    dinv_row, C = _prep_kernel(rowsum_row, colsum_row, Xg1)
    C2 = _gcn1_kernel(Wmm, C, dinv_row, Wg2)
    embs_graph = _gcn2_kernel(Wmm, C2, dinv_row)

    A, embs_hom = _sc_knn(OS, idx, beta16, alpha16)
    return (embs_hom, embs_graph, A, Y)


# speedup vs baseline: 8.7089x; 1.0169x over previous
import functools

import jax
import jax.numpy as jnp
from jax import lax
from jax.experimental import pallas as pl
from jax.experimental.pallas import tpu as pltpu
from jax.experimental.pallas import tpu_sc as plsc

N = 4096
D_FEAT = 256
OUT_FEAT = 64
HID = 512
GCN_HID = 512
GCN_OUT = 256
N_NEIGHBORS = 30
SCALE_K = 15
K = 10

F32 = jnp.float32
BM = 256
BMS = 128


def _dot(a, b):
    return jnp.dot(a.astype(jnp.bfloat16), b.astype(jnp.bfloat16),
                   preferred_element_type=F32)


def _dg(a, b, dims):
    return lax.dot_general(a.astype(jnp.bfloat16), b.astype(jnp.bfloat16),
                           (dims, ((), ())), preferred_element_type=F32)


def _dg_hi(a, b, dims):
    return lax.dot_general(a, b, (dims, ((), ())), preferred_element_type=F32,
                           precision=lax.Precision.HIGHEST)


def _gram_body(xo_ref, w1_ref, b1_ref, w2_ref, b2_ref, g_ref):
    i = pl.program_id(0)
    h = jnp.maximum(_dot(xo_ref[...], w1_ref[...]) + b1_ref[...], 0.0)
    yb = _dot(h, w2_ref[...]) + b2_ref[...]
    g = _dg(yb, yb, ((0,), (0,)))

    @pl.when(i == 0)
    def _():
        g_ref[...] = jnp.zeros_like(g_ref)

    g_ref[...] += g


def _gram(x_orth, W1, b1, W2, b2):
    return pl.pallas_call(
        _gram_body,
        grid=(N // BM,),
        in_specs=[
            pl.BlockSpec((BM, D_FEAT), lambda i: (i, 0)),
            pl.BlockSpec((D_FEAT, HID), lambda i: (0, 0)),
            pl.BlockSpec((1, HID), lambda i: (0, 0)),
            pl.BlockSpec((HID, OUT_FEAT), lambda i: (0, 0)),
            pl.BlockSpec((1, OUT_FEAT), lambda i: (0, 0)),
        ],
        out_specs=pl.BlockSpec((OUT_FEAT, OUT_FEAT), lambda i: (0, 0)),
        out_shape=jax.ShapeDtypeStruct((OUT_FEAT, OUT_FEAT), F32),
    )(x_orth, W1, b1, W2, b2)


def _chol_body(g_ref, li_ref):
    f = OUT_FEAT
    rows = lax.broadcasted_iota(jnp.int32, (f, f), 0)
    cols = lax.broadcasted_iota(jnp.int32, (f, f), 1)
    eye = (rows == cols).astype(F32)
    m = g_ref[...] / jnp.float32(N) + 1e-6 * eye

    def chol_step(j, L):
        r = m - _dg_hi(L, L, ((1,), (1,)))
        vcol = jnp.sum(jnp.where(cols == j, r, 0.0), axis=1, keepdims=True)
        dj = jnp.sum(jnp.where((rows == j) & (cols == j), r, 0.0))
        newcol = vcol / jnp.sqrt(dj)
        return jnp.where((cols == j) & (rows >= j), newcol, L)

    L = lax.fori_loop(0, f, chol_step, jnp.zeros((f, f), F32))

    colid = lax.broadcasted_iota(jnp.int32, (1, f), 1)

    def inv_step(j, Li):
        lrow = jnp.sum(jnp.where(rows == j, L, 0.0), axis=0, keepdims=True)
        prod = _dg_hi(lrow, Li, ((1,), (0,)))
        ljj = jnp.sum(jnp.where((rows == j) & (cols == j), L, 0.0))
        ej = (colid == j).astype(F32)
        newrow = (ej - prod) / ljj
        return jnp.where(rows == j, newrow, Li)

    li_ref[...] = lax.fori_loop(0, f, inv_step, jnp.zeros((f, f), F32))


def _chol_inv(G):
    return pl.pallas_call(
        _chol_body,
        out_shape=jax.ShapeDtypeStruct((OUT_FEAT, OUT_FEAT), F32),
    )(G)


def _mlp_body(x_ref, w1_ref, b1_ref, w2_ref, b2_ref, li_ref, wg1_ref,
              os_ref, y_ref, xg_ref):
    xb = x_ref[...]
    h = jnp.maximum(_dot(xb, w1_ref[...]) + b1_ref[...], 0.0)
    s = _dot(h, w2_ref[...]) + b2_ref[...]
    o = _dg(s, li_ref[...], ((1,), (1,)))
    os_ref[...] = jnp.concatenate([o, s], axis=1)
    y_ref[...] = 64.0 * o
    xg_ref[...] = _dot(xb, wg1_ref[...])


def _mlp_all(x, W1, b1, W2, b2, Linv, Wg1):
    return pl.pallas_call(
        _mlp_body,
        grid=(N // BM,),
        in_specs=[
            pl.BlockSpec((BM, D_FEAT), lambda i: (i, 0)),
            pl.BlockSpec((D_FEAT, HID), lambda i: (0, 0)),
            pl.BlockSpec((1, HID), lambda i: (0, 0)),
            pl.BlockSpec((HID, OUT_FEAT), lambda i: (0, 0)),
            pl.BlockSpec((1, OUT_FEAT), lambda i: (0, 0)),
            pl.BlockSpec((OUT_FEAT, OUT_FEAT), lambda i: (0, 0)),
            pl.BlockSpec((D_FEAT, GCN_HID), lambda i: (0, 0)),
        ],
        out_specs=[
            pl.BlockSpec((BM, 2 * OUT_FEAT), lambda i: (i, 0)),
            pl.BlockSpec((BM, OUT_FEAT), lambda i: (i, 0)),
            pl.BlockSpec((BM, GCN_HID), lambda i: (i, 0)),
        ],
        out_shape=[
            jax.ShapeDtypeStruct((N, 2 * OUT_FEAT), F32),
            jax.ShapeDtypeStruct((N, OUT_FEAT), F32),
            jax.ShapeDtypeStruct((N, GCN_HID), F32),
        ],
    )(x, W1, b1, W2, b2, Linv, Wg1)


def _kth_extract(mat, k, kind):
    rows = mat.shape[0]
    bits = lax.bitcast_convert_type(mat, jnp.int32)
    kf = jnp.float32(k)
    if kind == "min":
        lo = jnp.full((rows, 1), -1, jnp.int32)
        hi = jnp.full((rows, 1), 0x7F7FFFFF, jnp.int32)

        def step(_, carry):
            lo, hi = carry
            mid = lo + ((hi - lo) >> 1)
            cnt = jnp.sum((bits <= mid).astype(F32), axis=1, keepdims=True)
            pred = cnt >= kf
            return jnp.where(pred, lo, mid), jnp.where(pred, mid, hi)

        lo, hi = lax.fori_loop(0, 31, step, (lo, hi))
        ans = hi
    else:
        lo = jnp.zeros((rows, 1), jnp.int32)
        hi = jnp.full((rows, 1), 0x7F800000, jnp.int32)

        def step(_, carry):
            lo, hi = carry
            mid = lo + ((hi - lo) >> 1)
            cnt = jnp.sum((bits >= mid).astype(F32), axis=1, keepdims=True)
            pred = cnt >= kf
            return jnp.where(pred, mid, lo), jnp.where(pred, hi, mid)

        lo, hi = lax.fori_loop(0, 31, step, (lo, hi))
        ans = lo
    return lax.bitcast_convert_type(ans, F32)


def _d2_panel(xb, xf):
    sqb = jnp.sum(xb * xb, axis=1, keepdims=True)
    sqf = jnp.sum(xf * xf, axis=1)
    panel = _dg(xb, xf, ((1,), (1,)))
    return jnp.maximum(sqb + sqf[None, :] - 2.0 * panel, 0.0)


def _col_to_row(col):
    n = col.shape[0]
    r = lax.broadcasted_iota(jnp.int32, (n, n), 0)
    c = lax.broadcasted_iota(jnp.int32, (n, n), 1)
    eye = (r == c).astype(F32)
    return _dg_hi(col, eye, ((0,), (0,)))


def _row_to_col(row):
    n = row.shape[1]
    r = lax.broadcasted_iota(jnp.int32, (n, n), 0)
    c = lax.broadcasted_iota(jnp.int32, (n, n), 1)
    eye = (r == c).astype(F32)
    return _dg_hi(eye, row, ((0,), (1,)))


def _scale_body(x_ref, xf_ref, sc_ref, sr_ref):
    d2 = _d2_panel(x_ref[...], xf_ref[...])
    t = _kth_extract(d2, SCALE_K + 1, "min")
    s = jnp.sqrt(t + 1e-12)
    sc_ref[...] = s
    sr_ref[...] = _col_to_row(s)


def _scale_kernel(x):
    return pl.pallas_call(
        _scale_body,
        grid=(N // BMS,),
        in_specs=[
            pl.BlockSpec((BMS, D_FEAT), lambda i: (i, 0)),
            pl.BlockSpec((N, D_FEAT), lambda i: (0, 0)),
        ],
        out_specs=[
            pl.BlockSpec((BMS, 1), lambda i: (i, 0)),
            pl.BlockSpec((1, BMS), lambda i: (0, i)),
        ],
        out_shape=[
            jax.ShapeDtypeStruct((N, 1), F32),
            jax.ShapeDtypeStruct((1, N), F32),
        ],
    )(x, x)


def _wm_body(x_ref, xf_ref, sc_ref, sr_ref, wm_ref, rs_ref, cs_ref):
    i = pl.program_id(0)
    d2 = _d2_panel(x_ref[...], xf_ref[...])
    wm = jnp.exp(-d2 / (sc_ref[...] * sr_ref[...] + 1e-8))
    thr = _kth_extract(wm, N_NEIGHBORS + 1, "max")
    wmm = wm * (wm >= thr).astype(F32)
    wm_ref[...] = wmm.astype(jnp.bfloat16)
    rs = jnp.sum(wmm, axis=1, keepdims=True)
    rs_ref[...] = _col_to_row(rs)

    @pl.when(i == 0)
    def _():
        cs_ref[...] = jnp.zeros_like(cs_ref)

    cs_ref[...] += jnp.sum(wmm, axis=0, keepdims=True)


def _wm_kernel(x, scale_col, scale_row):
    return pl.pallas_call(
        _wm_body,
        grid=(N // BMS,),
        in_specs=[
            pl.BlockSpec((BMS, D_FEAT), lambda i: (i, 0)),
            pl.BlockSpec((N, D_FEAT), lambda i: (0, 0)),
            pl.BlockSpec((BMS, 1), lambda i: (i, 0)),
            pl.BlockSpec((1, N), lambda i: (0, 0)),
        ],
        out_specs=[
            pl.BlockSpec((BMS, N), lambda i: (i, 0)),
            pl.BlockSpec((1, BMS), lambda i: (0, i)),
            pl.BlockSpec((1, N), lambda i: (0, 0)),
        ],
        out_shape=[
            jax.ShapeDtypeStruct((N, N), jnp.bfloat16),
            jax.ShapeDtypeStruct((1, N), F32),
            jax.ShapeDtypeStruct((1, N), F32),
        ],
    )(x, x, scale_col, scale_row)


def _prep_body(rs_ref, cs_ref, xg_ref, dinv_ref, c_ref):
    i = pl.program_id(0)
    drow = 0.5 * (rs_ref[...] + cs_ref[...])
    dinv = 1.0 / jnp.sqrt(drow + 1e-8)
    dinv_ref[...] = dinv
    c_ref[...] = xg_ref[...] * _row_to_col(dinv)


def _prep_kernel(rowsum_row, colsum_row, Xg1):
    return pl.pallas_call(
        _prep_body,
        grid=(N // BM,),
        in_specs=[
            pl.BlockSpec((1, BM), lambda i: (0, i)),
            pl.BlockSpec((1, BM), lambda i: (0, i)),
            pl.BlockSpec((BM, GCN_HID), lambda i: (i, 0)),
        ],
        out_specs=[
            pl.BlockSpec((1, BM), lambda i: (0, i)),
            pl.BlockSpec((BM, GCN_HID), lambda i: (i, 0)),
        ],
        out_shape=[
            jax.ShapeDtypeStruct((1, N), F32),
            jax.ShapeDtypeStruct((N, GCN_HID), F32),
        ],
    )(rowsum_row, colsum_row, Xg1)


def _gcn1_body(wr_ref, wc_ref, c_ref, dv_ref, wg2_ref, c2_ref):
    dcol = _row_to_col(dv_ref[...])
    t1 = _dot(wr_ref[...], c_ref[...])
    t2 = _dg(wc_ref[...], c_ref[...], ((0,), (0,)))
    h = jnp.maximum(0.5 * dcol * (t1 + t2), 0.0)
    c2_ref[...] = dcol * _dot(h, wg2_ref[...])


def _gcn1_kernel(Wmm, C, dinv_row, Wg2):
    return pl.pallas_call(
        _gcn1_body,
        grid=(N // BM,),
        in_specs=[
            pl.BlockSpec((BM, N), lambda i: (i, 0)),
            pl.BlockSpec((N, BM), lambda i: (0, i)),
            pl.BlockSpec((N, GCN_HID), lambda i: (0, 0)),
            pl.BlockSpec((1, BM), lambda i: (0, i)),
            pl.BlockSpec((GCN_HID, GCN_OUT), lambda i: (0, 0)),
        ],
        out_specs=pl.BlockSpec((BM, GCN_OUT), lambda i: (i, 0)),
        out_shape=jax.ShapeDtypeStruct((N, GCN_OUT), F32),
    )(Wmm, Wmm, C, dinv_row, Wg2)


def _gcn2_body(wr_ref, wc_ref, c2_ref, dv_ref, out_ref):
    dcol = _row_to_col(dv_ref[...])
    t1 = _dot(wr_ref[...], c2_ref[...])
    t2 = _dg(wc_ref[...], c2_ref[...], ((0,), (0,)))
    out_ref[...] = 0.5 * dcol * (t1 + t2)


def _gcn2_kernel(Wmm, C2, dinv_row):
    return pl.pallas_call(
        _gcn2_body,
        grid=(N // BM,),
        in_specs=[
            pl.BlockSpec((BM, N), lambda i: (i, 0)),
            pl.BlockSpec((N, BM), lambda i: (0, i)),
            pl.BlockSpec((N, GCN_OUT), lambda i: (0, 0)),
            pl.BlockSpec((1, BM), lambda i: (0, i)),
        ],
        out_specs=pl.BlockSpec((BM, GCN_OUT), lambda i: (i, 0)),
        out_shape=jax.ShapeDtypeStruct((N, GCN_OUT), F32),
    )(Wmm, Wmm, C2, dinv_row)


ROWS_PW = N // 32
CH = 8
NCHUNK = ROWS_PW // CH


def _sc_sqrt(x):
    i = plsc.bitcast(x, jnp.int32)
    r = plsc.bitcast(jnp.int32(0x5F3759DF) - (i >> 1), F32)
    for _ in range(3):
        r = r * (1.5 - 0.5 * x * r * r)
    s = x * r
    return 0.5 * (s + x / s)


def _sc_knn_body(os_hbm, idx_hbm, idxf_hbm, bet_hbm, alp_hbm,
                 a_hbm, eh_hbm,
                 idx_v, idx_f, selfo, g_os, stripe, ebuf, ib,
                 scal, sem_g, sem_s):
    wid = lax.axis_index("s") * 2 + lax.axis_index("c")
    base = wid * ROWS_PW

    pltpu.sync_copy(idx_hbm.at[pl.ds(base, ROWS_PW)], idx_v)
    pltpu.sync_copy(idxf_hbm.at[pl.ds(base * (K + 6), ROWS_PW * (K + 6))],
                    idx_f)
    pltpu.sync_copy(bet_hbm, scal)
    betav = scal[...]
    pltpu.sync_copy(alp_hbm, scal)
    alpav = scal[...]

    ii = lax.broadcasted_iota(jnp.int32, (16,), 0)
    valid = (ii >= 1) & (ii <= K)
    zeros16 = jnp.zeros((16,), F32)
    jf = (ii + 1).astype(F32)

    def zstep(t, _):
        stripe[t // (N // 16), pl.ds((t % (N // 16)) * 16, 16)] = zeros16
        return 0
    lax.fori_loop(0, CH * (N // 16), zstep, 0)

    def chunk_step(c, _):
        row0 = base + c * CH
        gcp = pltpu.async_copy(os_hbm.at[idx_f.at[pl.ds(c * CH * 16, CH * 16)]],
                               g_os, sem_g)
        pltpu.sync_copy(os_hbm.at[pl.ds(row0, CH)], selfo)
        gcp.wait()

        def row_step(r, _):
            i0 = c * CH + r
            idxvec = idx_v[i0]
            d2 = zeros16
            for j in range(1, K + 1):
                acc = zeros16
                for q in range(4):
                    t = g_os[r * 16 + j, pl.ds(q * 16, 16)] - selfo[r, pl.ds(q * 16, 16)]
                    acc = acc + t * t
                d2 = jnp.where(ii == j, jnp.sum(acc), d2)
            dxi = _sc_sqrt(d2 + 1e-8)
            dfi = _sc_sqrt(4096.0 * d2 + 1e-12)
            ad = -(dxi + betav * dfi) / (2.0 * alpav)
            adm = jnp.where(valid, ad, -3.0e38)
            u, _unused = plsc.sort_key_val(adm, adm, descending=True)
            css = plsc.cumsum(u)
            cond = ((u + (1.0 - css) / jf) > 0) & (ii < K)
            rho = plsc.all_reduce_population_count(cond)
            rhof = rho.astype(F32)
            cssrho = jnp.sum(jnp.where(ii == rho - 1, css, 0.0))
            theta = (cssrho - 1.0) / rhof
            vals = jnp.where(valid, jnp.maximum(ad - theta, 0.0), 0.0)
            ib[...] = idxvec
            dup = ii < 0
            for s in range(1, K):
                sh = plsc.load_gather(ib, [jnp.minimum(ii + s, 15)])
                dup = dup | ((idxvec == sh) & (ii >= 1) & (ii + s <= K))
            wvals = jnp.where(valid & (~dup), vals, 0.0)
            rsplat = jnp.broadcast_to(r, (16,)).astype(jnp.int32)
            plsc.store_scatter(stripe, [rsplat, idxvec], vals, mask=valid)
            ib[...] = plsc.bitcast(wvals, jnp.int32)
            accs = [zeros16] * 4
            for j in range(1, K + 1):
                wj = plsc.bitcast(
                    plsc.load_gather(ib, [jnp.full((16,), j, jnp.int32)]), F32)
                for q in range(4):
                    accs[q] = accs[q] + wj * g_os[r * 16 + j, pl.ds(64 + q * 16, 16)]
            for q in range(4):
                ebuf[i0, pl.ds(q * 16, 16)] = accs[q]
            return 0

        lax.fori_loop(0, CH, row_step, 0)
        pltpu.sync_copy(stripe, a_hbm.at[pl.ds(row0, CH)])

        def rz_step(r, _):
            idxvec = idx_v[c * CH + r]
            rsplat = jnp.broadcast_to(r, (16,)).astype(jnp.int32)
            plsc.store_scatter(stripe, [rsplat, idxvec], zeros16, mask=valid)
            return 0
        lax.fori_loop(0, CH, rz_step, 0)
        return 0

    lax.fori_loop(0, NCHUNK, chunk_step, 0)
    pltpu.sync_copy(ebuf, eh_hbm.at[pl.ds(base, ROWS_PW)])


def _sc_knn(OS, idx, beta16, alpha16):
    mesh = plsc.VectorSubcoreMesh(core_axis_name="c", subcore_axis_name="s")
    f = functools.partial(
        pl.kernel,
        out_type=[
            jax.ShapeDtypeStruct((N, N), F32),
            jax.ShapeDtypeStruct((N, OUT_FEAT), F32),
        ],
        mesh=mesh,
        compiler_params=pltpu.CompilerParams(needs_layout_passes=False),
        scratch_types=[
            pltpu.VMEM((ROWS_PW, K + 6), jnp.int32),
            pltpu.VMEM((ROWS_PW * (K + 6),), jnp.int32),
            pltpu.VMEM((CH, 2 * OUT_FEAT), F32),
            pltpu.VMEM((CH * 16, 2 * OUT_FEAT), F32),
            pltpu.VMEM((CH, N), F32),
            pltpu.VMEM((ROWS_PW, OUT_FEAT), F32),
            pltpu.VMEM((16,), jnp.int32),
            pltpu.VMEM((16,), F32),
            pltpu.SemaphoreType.DMA,
            pltpu.SemaphoreType.DMA,
        ],
    )(_sc_knn_body)
    idx_flat = idx.reshape(N * (K + 6))
    return f(OS, idx, idx_flat, beta16, alpha16)


def kernel(x, x_orth, beta, alpha, idx, W1, b1, W2, b2, Wg1, Wg2):
    b1r = b1.reshape(1, HID)
    b2r = b2.reshape(1, OUT_FEAT)
    idx = idx.astype(jnp.int32)
    beta16 = jnp.full((16,), beta, F32)
    alpha16 = jnp.full((16,), alpha, F32)

    G = _gram(x_orth, W1, b1r, W2, b2r)
    Linv = _chol_inv(G)
    OS, Y, Xg1 = _mlp_all(x, W1, b1r, W2, b2r, Linv, Wg1)

    scale_col, scale_row = _scale_kernel(x)
    Wmm, rowsum_row, colsum_row = _wm_kernel(x, scale_col, scale_row)
    dinv_row, C = _prep_kernel(rowsum_row, colsum_row, Xg1)
    C2 = _gcn1_kernel(Wmm, C, dinv_row, Wg2)
    embs_graph = _gcn2_kernel(Wmm, C2, dinv_row)

    A, embs_hom = _sc_knn(OS, idx, beta16, alpha16)
    return (embs_hom, embs_graph, A, Y)


# speedup vs baseline: 9.9398x; 1.1413x over previous
import functools

import jax
import jax.numpy as jnp
from jax import lax
from jax.experimental import pallas as pl
from jax.experimental.pallas import tpu as pltpu
from jax.experimental.pallas import tpu_sc as plsc

N = 4096
D_FEAT = 256
OUT_FEAT = 64
HID = 512
GCN_HID = 512
GCN_OUT = 256
N_NEIGHBORS = 30
SCALE_K = 15
K = 10

F32 = jnp.float32
BM = 256
BMS = 256


def _dot(a, b):
    return jnp.dot(a.astype(jnp.bfloat16), b.astype(jnp.bfloat16),
                   preferred_element_type=F32)


def _dg(a, b, dims):
    return lax.dot_general(a.astype(jnp.bfloat16), b.astype(jnp.bfloat16),
                           (dims, ((), ())), preferred_element_type=F32)


def _dg_hi(a, b, dims):
    return lax.dot_general(a, b, (dims, ((), ())), preferred_element_type=F32,
                           precision=lax.Precision.HIGHEST)


def _gram_body(xo_ref, w1_ref, b1_ref, w2_ref, b2_ref, g_ref):
    i = pl.program_id(0)
    h = jnp.maximum(_dot(xo_ref[...], w1_ref[...]) + b1_ref[...], 0.0)
    yb = _dot(h, w2_ref[...]) + b2_ref[...]
    g = _dg(yb, yb, ((0,), (0,)))

    @pl.when(i == 0)
    def _():
        g_ref[...] = jnp.zeros_like(g_ref)

    g_ref[...] += g


def _gram(x_orth, W1, b1, W2, b2):
    return pl.pallas_call(
        _gram_body,
        grid=(N // BM,),
        in_specs=[
            pl.BlockSpec((BM, D_FEAT), lambda i: (i, 0)),
            pl.BlockSpec((D_FEAT, HID), lambda i: (0, 0)),
            pl.BlockSpec((1, HID), lambda i: (0, 0)),
            pl.BlockSpec((HID, OUT_FEAT), lambda i: (0, 0)),
            pl.BlockSpec((1, OUT_FEAT), lambda i: (0, 0)),
        ],
        out_specs=pl.BlockSpec((OUT_FEAT, OUT_FEAT), lambda i: (0, 0)),
        out_shape=jax.ShapeDtypeStruct((OUT_FEAT, OUT_FEAT), F32),
    )(x_orth, W1, b1, W2, b2)


def _chol_body(g_ref, li_ref):
    f = OUT_FEAT
    rows = lax.broadcasted_iota(jnp.int32, (f, f), 0)
    cols = lax.broadcasted_iota(jnp.int32, (f, f), 1)
    eye = (rows == cols).astype(F32)
    m = g_ref[...] / jnp.float32(N) + 1e-6 * eye

    def chol_step(j, L):
        r = m - _dg_hi(L, L, ((1,), (1,)))
        vcol = jnp.sum(jnp.where(cols == j, r, 0.0), axis=1, keepdims=True)
        dj = jnp.sum(jnp.where((rows == j) & (cols == j), r, 0.0))
        newcol = vcol / jnp.sqrt(dj)
        return jnp.where((cols == j) & (rows >= j), newcol, L)

    L = lax.fori_loop(0, f, chol_step, jnp.zeros((f, f), F32))

    colid = lax.broadcasted_iota(jnp.int32, (1, f), 1)

    def inv_step(j, Li):
        lrow = jnp.sum(jnp.where(rows == j, L, 0.0), axis=0, keepdims=True)
        prod = _dg_hi(lrow, Li, ((1,), (0,)))
        ljj = jnp.sum(jnp.where((rows == j) & (cols == j), L, 0.0))
        ej = (colid == j).astype(F32)
        newrow = (ej - prod) / ljj
        return jnp.where(rows == j, newrow, Li)

    li_ref[...] = lax.fori_loop(0, f, inv_step, jnp.zeros((f, f), F32))


def _chol_inv(G):
    return pl.pallas_call(
        _chol_body,
        out_shape=jax.ShapeDtypeStruct((OUT_FEAT, OUT_FEAT), F32),
    )(G)


def _mlp_body(x_ref, w1_ref, b1_ref, w2_ref, b2_ref, li_ref, wg1_ref,
              os_ref, y_ref, xg_ref):
    xb = x_ref[...]
    h = jnp.maximum(_dot(xb, w1_ref[...]) + b1_ref[...], 0.0)
    s = _dot(h, w2_ref[...]) + b2_ref[...]
    o = _dg(s, li_ref[...], ((1,), (1,)))
    os_ref[...] = jnp.concatenate([o, s], axis=1)
    y_ref[...] = 64.0 * o
    xg_ref[...] = _dot(xb, wg1_ref[...])


def _mlp_all(x, W1, b1, W2, b2, Linv, Wg1):
    return pl.pallas_call(
        _mlp_body,
        grid=(N // BM,),
        in_specs=[
            pl.BlockSpec((BM, D_FEAT), lambda i: (i, 0)),
            pl.BlockSpec((D_FEAT, HID), lambda i: (0, 0)),
            pl.BlockSpec((1, HID), lambda i: (0, 0)),
            pl.BlockSpec((HID, OUT_FEAT), lambda i: (0, 0)),
            pl.BlockSpec((1, OUT_FEAT), lambda i: (0, 0)),
            pl.BlockSpec((OUT_FEAT, OUT_FEAT), lambda i: (0, 0)),
            pl.BlockSpec((D_FEAT, GCN_HID), lambda i: (0, 0)),
        ],
        out_specs=[
            pl.BlockSpec((BM, 2 * OUT_FEAT), lambda i: (i, 0)),
            pl.BlockSpec((BM, OUT_FEAT), lambda i: (i, 0)),
            pl.BlockSpec((BM, GCN_HID), lambda i: (i, 0)),
        ],
        out_shape=[
            jax.ShapeDtypeStruct((N, 2 * OUT_FEAT), F32),
            jax.ShapeDtypeStruct((N, OUT_FEAT), F32),
            jax.ShapeDtypeStruct((N, GCN_HID), F32),
        ],
    )(x, W1, b1, W2, b2, Linv, Wg1)


def _kth_extract(mat, k, kind):
    rows = mat.shape[0]
    bits = lax.bitcast_convert_type(mat, jnp.int32)
    kf = jnp.float32(k)
    if kind == "min":
        lo = jnp.full((rows, 1), -1, jnp.int32)
        hi = jnp.full((rows, 1), 0x7F7FFFFF, jnp.int32)

        def step(_, carry):
            lo, hi = carry
            mid = lo + ((hi - lo) >> 1)
            cnt = jnp.sum((bits <= mid).astype(F32), axis=1, keepdims=True)
            pred = cnt >= kf
            return jnp.where(pred, lo, mid), jnp.where(pred, mid, hi)

        lo, hi = lax.fori_loop(0, 31, step, (lo, hi))
        ans = hi
    else:
        lo = jnp.zeros((rows, 1), jnp.int32)
        hi = jnp.full((rows, 1), 0x7F800000, jnp.int32)

        def step(_, carry):
            lo, hi = carry
            mid = lo + ((hi - lo) >> 1)
            cnt = jnp.sum((bits >= mid).astype(F32), axis=1, keepdims=True)
            pred = cnt >= kf
            return jnp.where(pred, mid, lo), jnp.where(pred, hi, mid)

        lo, hi = lax.fori_loop(0, 31, step, (lo, hi))
        ans = lo
    return lax.bitcast_convert_type(ans, F32)


def _d2_panel(xb, xf):
    sqb = jnp.sum(xb * xb, axis=1, keepdims=True)
    sqf = jnp.sum(xf * xf, axis=1)
    panel = _dg(xb, xf, ((1,), (1,)))
    return jnp.maximum(sqb + sqf[None, :] - 2.0 * panel, 0.0)


def _col_to_row(col):
    n = col.shape[0]
    r = lax.broadcasted_iota(jnp.int32, (n, n), 0)
    c = lax.broadcasted_iota(jnp.int32, (n, n), 1)
    eye = (r == c).astype(F32)
    return _dg_hi(col, eye, ((0,), (0,)))


def _row_to_col(row):
    n = row.shape[1]
    r = lax.broadcasted_iota(jnp.int32, (n, n), 0)
    c = lax.broadcasted_iota(jnp.int32, (n, n), 1)
    eye = (r == c).astype(F32)
    return _dg_hi(eye, row, ((0,), (1,)))


def _scale_body(x_ref, xf_ref, sc_ref, sr_ref):
    d2 = _d2_panel(x_ref[...], xf_ref[...])
    t = _kth_extract(d2, SCALE_K + 1, "min")
    s = jnp.sqrt(t + 1e-12)
    sc_ref[...] = s
    sr_ref[...] = _col_to_row(s)


def _scale_kernel(x):
    return pl.pallas_call(
        _scale_body,
        grid=(N // BMS,),
        in_specs=[
            pl.BlockSpec((BMS, D_FEAT), lambda i: (i, 0)),
            pl.BlockSpec((N, D_FEAT), lambda i: (0, 0)),
        ],
        out_specs=[
            pl.BlockSpec((BMS, 1), lambda i: (i, 0)),
            pl.BlockSpec((1, BMS), lambda i: (0, i)),
        ],
        out_shape=[
            jax.ShapeDtypeStruct((N, 1), F32),
            jax.ShapeDtypeStruct((1, N), F32),
        ],
    )(x, x)


def _wm_body(x_ref, xf_ref, sc_ref, sr_ref, wm_ref, rs_ref, cs_ref):
    i = pl.program_id(0)
    d2 = _d2_panel(x_ref[...], xf_ref[...])
    wm = jnp.exp(-d2 / (sc_ref[...] * sr_ref[...] + 1e-8))
    thr = _kth_extract(wm, N_NEIGHBORS + 1, "max")
    wmm = wm * (wm >= thr).astype(F32)
    wm_ref[...] = wmm.astype(jnp.bfloat16)
    rs = jnp.sum(wmm, axis=1, keepdims=True)
    rs_ref[...] = _col_to_row(rs)

    @pl.when(i == 0)
    def _():
        cs_ref[...] = jnp.zeros_like(cs_ref)

    cs_ref[...] += jnp.sum(wmm, axis=0, keepdims=True)


def _wm_kernel(x, scale_col, scale_row):
    return pl.pallas_call(
        _wm_body,
        grid=(N // BMS,),
        in_specs=[
            pl.BlockSpec((BMS, D_FEAT), lambda i: (i, 0)),
            pl.BlockSpec((N, D_FEAT), lambda i: (0, 0)),
            pl.BlockSpec((BMS, 1), lambda i: (i, 0)),
            pl.BlockSpec((1, N), lambda i: (0, 0)),
        ],
        out_specs=[
            pl.BlockSpec((BMS, N), lambda i: (i, 0)),
            pl.BlockSpec((1, BMS), lambda i: (0, i)),
            pl.BlockSpec((1, N), lambda i: (0, 0)),
        ],
        out_shape=[
            jax.ShapeDtypeStruct((N, N), jnp.bfloat16),
            jax.ShapeDtypeStruct((1, N), F32),
            jax.ShapeDtypeStruct((1, N), F32),
        ],
    )(x, x, scale_col, scale_row)


def _prep_body(rs_ref, cs_ref, xg_ref, dinv_ref, c_ref):
    i = pl.program_id(0)
    drow = 0.5 * (rs_ref[...] + cs_ref[...])
    dinv = 1.0 / jnp.sqrt(drow + 1e-8)
    dinv_ref[...] = dinv
    c_ref[...] = xg_ref[...] * _row_to_col(dinv)


def _prep_kernel(rowsum_row, colsum_row, Xg1):
    return pl.pallas_call(
        _prep_body,
        grid=(N // BM,),
        in_specs=[
            pl.BlockSpec((1, BM), lambda i: (0, i)),
            pl.BlockSpec((1, BM), lambda i: (0, i)),
            pl.BlockSpec((BM, GCN_HID), lambda i: (i, 0)),
        ],
        out_specs=[
            pl.BlockSpec((1, BM), lambda i: (0, i)),
            pl.BlockSpec((BM, GCN_HID), lambda i: (i, 0)),
        ],
        out_shape=[
            jax.ShapeDtypeStruct((1, N), F32),
            jax.ShapeDtypeStruct((N, GCN_HID), F32),
        ],
    )(rowsum_row, colsum_row, Xg1)


def _gcn1_body(wr_ref, wc_ref, c_ref, dv_ref, wg2_ref, c2_ref):
    dcol = _row_to_col(dv_ref[...])
    t1 = _dot(wr_ref[...], c_ref[...])
    t2 = _dg(wc_ref[...], c_ref[...], ((0,), (0,)))
    h = jnp.maximum(0.5 * dcol * (t1 + t2), 0.0)
    c2_ref[...] = dcol * _dot(h, wg2_ref[...])


def _gcn1_kernel(Wmm, C, dinv_row, Wg2):
    return pl.pallas_call(
        _gcn1_body,
        grid=(N // BM,),
        in_specs=[
            pl.BlockSpec((BM, N), lambda i: (i, 0)),
            pl.BlockSpec((N, BM), lambda i: (0, i)),
            pl.BlockSpec((N, GCN_HID), lambda i: (0, 0)),
            pl.BlockSpec((1, BM), lambda i: (0, i)),
            pl.BlockSpec((GCN_HID, GCN_OUT), lambda i: (0, 0)),
        ],
        out_specs=pl.BlockSpec((BM, GCN_OUT), lambda i: (i, 0)),
        out_shape=jax.ShapeDtypeStruct((N, GCN_OUT), F32),
    )(Wmm, Wmm, C, dinv_row, Wg2)


def _gcn2_body(wr_ref, wc_ref, c2_ref, dv_ref, out_ref):
    dcol = _row_to_col(dv_ref[...])
    t1 = _dot(wr_ref[...], c2_ref[...])
    t2 = _dg(wc_ref[...], c2_ref[...], ((0,), (0,)))
    out_ref[...] = 0.5 * dcol * (t1 + t2)


def _gcn2_kernel(Wmm, C2, dinv_row):
    return pl.pallas_call(
        _gcn2_body,
        grid=(N // BM,),
        in_specs=[
            pl.BlockSpec((BM, N), lambda i: (i, 0)),
            pl.BlockSpec((N, BM), lambda i: (0, i)),
            pl.BlockSpec((N, GCN_OUT), lambda i: (0, 0)),
            pl.BlockSpec((1, BM), lambda i: (0, i)),
        ],
        out_specs=pl.BlockSpec((BM, GCN_OUT), lambda i: (i, 0)),
        out_shape=jax.ShapeDtypeStruct((N, GCN_OUT), F32),
    )(Wmm, Wmm, C2, dinv_row)


ROWS_PW = N // 32
CH = 8
NCHUNK = ROWS_PW // CH


def _sc_sqrt(x):
    i = plsc.bitcast(x, jnp.int32)
    r = plsc.bitcast(jnp.int32(0x5F3759DF) - (i >> 1), F32)
    for _ in range(3):
        r = r * (1.5 - 0.5 * x * r * r)
    s = x * r
    return 0.5 * (s + x / s)


def _sc_knn_body(os_hbm, idx_hbm, idxf_hbm, bet_hbm, alp_hbm,
                 a_hbm, eh_hbm,
                 idx_v, idx_f, selfo, g_os, stripe, ebuf, ib,
                 scal, sem_g, sem_s):
    wid = lax.axis_index("s") * 2 + lax.axis_index("c")
    base = wid * ROWS_PW

    pltpu.sync_copy(idx_hbm.at[pl.ds(base, ROWS_PW)], idx_v)
    pltpu.sync_copy(idxf_hbm.at[pl.ds(base * (K + 6), ROWS_PW * (K + 6))],
                    idx_f)
    pltpu.sync_copy(bet_hbm, scal)
    betav = scal[...]
    pltpu.sync_copy(alp_hbm, scal)
    alpav = scal[...]

    ii = lax.broadcasted_iota(jnp.int32, (16,), 0)
    valid = (ii >= 1) & (ii <= K)
    zeros16 = jnp.zeros((16,), F32)
    jf = (ii + 1).astype(F32)

    def zstep(t, _):
        stripe[t // (N // 16), pl.ds((t % (N // 16)) * 16, 16)] = zeros16
        return 0
    lax.fori_loop(0, CH * (N // 16), zstep, 0)

    def chunk_step(c, _):
        row0 = base + c * CH
        gcp = pltpu.async_copy(os_hbm.at[idx_f.at[pl.ds(c * CH * 16, CH * 16)]],
                               g_os, sem_g)
        pltpu.sync_copy(os_hbm.at[pl.ds(row0, CH)], selfo)
        gcp.wait()

        def row_step(r, _):
            i0 = c * CH + r
            idxvec = idx_v[i0]
            d2 = zeros16
            for j in range(1, K + 1):
                acc = zeros16
                for q in range(4):
                    t = g_os[r * 16 + j, pl.ds(q * 16, 16)] - selfo[r, pl.ds(q * 16, 16)]
                    acc = acc + t * t
                d2 = jnp.where(ii == j, jnp.sum(acc), d2)
            dxi = _sc_sqrt(d2 + 1e-8)
            dfi = _sc_sqrt(4096.0 * d2 + 1e-12)
            ad = -(dxi + betav * dfi) / (2.0 * alpav)
            adm = jnp.where(valid, ad, -3.0e38)
            u, _unused = plsc.sort_key_val(adm, adm, descending=True)
            css = plsc.cumsum(u)
            cond = ((u + (1.0 - css) / jf) > 0) & (ii < K)
            rho = plsc.all_reduce_population_count(cond)
            rhof = rho.astype(F32)
            cssrho = jnp.sum(jnp.where(ii == rho - 1, css, 0.0))
            theta = (cssrho - 1.0) / rhof
            vals = jnp.where(valid, jnp.maximum(ad - theta, 0.0), 0.0)
            ib[...] = idxvec
            dup = ii < 0
            for s in range(1, K):
                sh = plsc.load_gather(ib, [jnp.minimum(ii + s, 15)])
                dup = dup | ((idxvec == sh) & (ii >= 1) & (ii + s <= K))
            wvals = jnp.where(valid & (~dup), vals, 0.0)
            rsplat = jnp.broadcast_to(r, (16,)).astype(jnp.int32)
            plsc.store_scatter(stripe, [rsplat, idxvec], vals, mask=valid)
            ib[...] = plsc.bitcast(wvals, jnp.int32)
            accs = [zeros16] * 4
            for j in range(1, K + 1):
                wj = plsc.bitcast(
                    plsc.load_gather(ib, [jnp.full((16,), j, jnp.int32)]), F32)
                for q in range(4):
                    accs[q] = accs[q] + wj * g_os[r * 16 + j, pl.ds(64 + q * 16, 16)]
            for q in range(4):
                ebuf[i0, pl.ds(q * 16, 16)] = accs[q]
            return 0

        lax.fori_loop(0, CH, row_step, 0)
        pltpu.sync_copy(stripe, a_hbm.at[pl.ds(row0, CH)])

        def rz_step(r, _):
            idxvec = idx_v[c * CH + r]
            rsplat = jnp.broadcast_to(r, (16,)).astype(jnp.int32)
            plsc.store_scatter(stripe, [rsplat, idxvec], zeros16, mask=valid)
            return 0
        lax.fori_loop(0, CH, rz_step, 0)
        return 0

    lax.fori_loop(0, NCHUNK, chunk_step, 0)
    pltpu.sync_copy(ebuf, eh_hbm.at[pl.ds(base, ROWS_PW)])


def _sc_knn(OS, idx, beta16, alpha16):
    mesh = plsc.VectorSubcoreMesh(core_axis_name="c", subcore_axis_name="s")
    f = functools.partial(
        pl.kernel,
        out_type=[
            jax.ShapeDtypeStruct((N, N), F32),
            jax.ShapeDtypeStruct((N, OUT_FEAT), F32),
        ],
        mesh=mesh,
        compiler_params=pltpu.CompilerParams(needs_layout_passes=False),
        scratch_types=[
            pltpu.VMEM((ROWS_PW, K + 6), jnp.int32),
            pltpu.VMEM((ROWS_PW * (K + 6),), jnp.int32),
            pltpu.VMEM((CH, 2 * OUT_FEAT), F32),
            pltpu.VMEM((CH * 16, 2 * OUT_FEAT), F32),
            pltpu.VMEM((CH, N), F32),
            pltpu.VMEM((ROWS_PW, OUT_FEAT), F32),
            pltpu.VMEM((16,), jnp.int32),
            pltpu.VMEM((16,), F32),
            pltpu.SemaphoreType.DMA,
            pltpu.SemaphoreType.DMA,
        ],
    )(_sc_knn_body)
    idx_flat = idx.reshape(N * (K + 6))
    return f(OS, idx, idx_flat, beta16, alpha16)


def kernel(x, x_orth, beta, alpha, idx, W1, b1, W2, b2, Wg1, Wg2):
    b1r = b1.reshape(1, HID)
    b2r = b2.reshape(1, OUT_FEAT)
    idx = idx.astype(jnp.int32)
    beta16 = jnp.full((16,), beta, F32)
    alpha16 = jnp.full((16,), alpha, F32)

    G = _gram(x_orth, W1, b1r, W2, b2r)
    Linv = _chol_inv(G)
    OS, Y, Xg1 = _mlp_all(x, W1, b1r, W2, b2r, Linv, Wg1)

    scale_col, scale_row = _scale_kernel(x)
    Wmm, rowsum_row, colsum_row = _wm_kernel(x, scale_col, scale_row)
    dinv_row, C = _prep_kernel(rowsum_row, colsum_row, Xg1)
    C2 = _gcn1_kernel(Wmm, C, dinv_row, Wg2)
    embs_graph = _gcn2_kernel(Wmm, C2, dinv_row)

    A, embs_hom = _sc_knn(OS, idx, beta16, alpha16)
    return (embs_hom, embs_graph, A, Y)


# speedup vs baseline: 10.6676x; 1.0732x over previous
import functools

import jax
import jax.numpy as jnp
from jax import lax
from jax.experimental import pallas as pl
from jax.experimental.pallas import tpu as pltpu
from jax.experimental.pallas import tpu_sc as plsc

N = 4096
D_FEAT = 256
OUT_FEAT = 64
HID = 512
GCN_HID = 512
GCN_OUT = 256
N_NEIGHBORS = 30
SCALE_K = 15
K = 10

F32 = jnp.float32
BM = 256
BMS = 512


def _dot(a, b):
    return jnp.dot(a.astype(jnp.bfloat16), b.astype(jnp.bfloat16),
                   preferred_element_type=F32)


def _dg(a, b, dims):
    return lax.dot_general(a.astype(jnp.bfloat16), b.astype(jnp.bfloat16),
                           (dims, ((), ())), preferred_element_type=F32)


def _dg_hi(a, b, dims):
    return lax.dot_general(a, b, (dims, ((), ())), preferred_element_type=F32,
                           precision=lax.Precision.HIGHEST)


def _gram_body(xo_ref, w1_ref, b1_ref, w2_ref, b2_ref, g_ref):
    i = pl.program_id(0)
    h = jnp.maximum(_dot(xo_ref[...], w1_ref[...]) + b1_ref[...], 0.0)
    yb = _dot(h, w2_ref[...]) + b2_ref[...]
    g = _dg(yb, yb, ((0,), (0,)))

    @pl.when(i == 0)
    def _():
        g_ref[...] = jnp.zeros_like(g_ref)

    g_ref[...] += g


def _gram(x_orth, W1, b1, W2, b2):
    return pl.pallas_call(
        _gram_body,
        grid=(N // BM,),
        in_specs=[
            pl.BlockSpec((BM, D_FEAT), lambda i: (i, 0)),
            pl.BlockSpec((D_FEAT, HID), lambda i: (0, 0)),
            pl.BlockSpec((1, HID), lambda i: (0, 0)),
            pl.BlockSpec((HID, OUT_FEAT), lambda i: (0, 0)),
            pl.BlockSpec((1, OUT_FEAT), lambda i: (0, 0)),
        ],
        out_specs=pl.BlockSpec((OUT_FEAT, OUT_FEAT), lambda i: (0, 0)),
        out_shape=jax.ShapeDtypeStruct((OUT_FEAT, OUT_FEAT), F32),
    )(x_orth, W1, b1, W2, b2)


def _chol_body(g_ref, li_ref):
    f = OUT_FEAT
    rows = lax.broadcasted_iota(jnp.int32, (f, f), 0)
    cols = lax.broadcasted_iota(jnp.int32, (f, f), 1)
    eye = (rows == cols).astype(F32)
    m = g_ref[...] / jnp.float32(N) + 1e-6 * eye

    def chol_step(j, L):
        r = m - _dg_hi(L, L, ((1,), (1,)))
        vcol = jnp.sum(jnp.where(cols == j, r, 0.0), axis=1, keepdims=True)
        dj = jnp.sum(jnp.where((rows == j) & (cols == j), r, 0.0))
        newcol = vcol / jnp.sqrt(dj)
        return jnp.where((cols == j) & (rows >= j), newcol, L)

    L = lax.fori_loop(0, f, chol_step, jnp.zeros((f, f), F32))

    colid = lax.broadcasted_iota(jnp.int32, (1, f), 1)

    def inv_step(j, Li):
        lrow = jnp.sum(jnp.where(rows == j, L, 0.0), axis=0, keepdims=True)
        prod = _dg_hi(lrow, Li, ((1,), (0,)))
        ljj = jnp.sum(jnp.where((rows == j) & (cols == j), L, 0.0))
        ej = (colid == j).astype(F32)
        newrow = (ej - prod) / ljj
        return jnp.where(rows == j, newrow, Li)

    li_ref[...] = lax.fori_loop(0, f, inv_step, jnp.zeros((f, f), F32))


def _chol_inv(G):
    return pl.pallas_call(
        _chol_body,
        out_shape=jax.ShapeDtypeStruct((OUT_FEAT, OUT_FEAT), F32),
    )(G)


def _mlp_body(x_ref, w1_ref, b1_ref, w2_ref, b2_ref, li_ref, wg1_ref,
              os_ref, y_ref, xg_ref):
    xb = x_ref[...]
    h = jnp.maximum(_dot(xb, w1_ref[...]) + b1_ref[...], 0.0)
    s = _dot(h, w2_ref[...]) + b2_ref[...]
    o = _dg(s, li_ref[...], ((1,), (1,)))
    os_ref[...] = jnp.concatenate([o, s], axis=1)
    y_ref[...] = 64.0 * o
    xg_ref[...] = _dot(xb, wg1_ref[...])


def _mlp_all(x, W1, b1, W2, b2, Linv, Wg1):
    return pl.pallas_call(
        _mlp_body,
        grid=(N // BM,),
        in_specs=[
            pl.BlockSpec((BM, D_FEAT), lambda i: (i, 0)),
            pl.BlockSpec((D_FEAT, HID), lambda i: (0, 0)),
            pl.BlockSpec((1, HID), lambda i: (0, 0)),
            pl.BlockSpec((HID, OUT_FEAT), lambda i: (0, 0)),
            pl.BlockSpec((1, OUT_FEAT), lambda i: (0, 0)),
            pl.BlockSpec((OUT_FEAT, OUT_FEAT), lambda i: (0, 0)),
            pl.BlockSpec((D_FEAT, GCN_HID), lambda i: (0, 0)),
        ],
        out_specs=[
            pl.BlockSpec((BM, 2 * OUT_FEAT), lambda i: (i, 0)),
            pl.BlockSpec((BM, OUT_FEAT), lambda i: (i, 0)),
            pl.BlockSpec((BM, GCN_HID), lambda i: (i, 0)),
        ],
        out_shape=[
            jax.ShapeDtypeStruct((N, 2 * OUT_FEAT), F32),
            jax.ShapeDtypeStruct((N, OUT_FEAT), F32),
            jax.ShapeDtypeStruct((N, GCN_HID), F32),
        ],
    )(x, W1, b1, W2, b2, Linv, Wg1)


def _kth_extract(mat, k, kind):
    rows = mat.shape[0]
    bits = lax.bitcast_convert_type(mat, jnp.int32)
    kf = jnp.float32(k)
    if kind == "min":
        lo = jnp.full((rows, 1), -1, jnp.int32)
        hi = jnp.full((rows, 1), 0x7F7FFFFF, jnp.int32)

        def step(_, carry):
            lo, hi = carry
            mid = lo + ((hi - lo) >> 1)
            cnt = jnp.sum((bits <= mid).astype(F32), axis=1, keepdims=True)
            pred = cnt >= kf
            return jnp.where(pred, lo, mid), jnp.where(pred, mid, hi)

        lo, hi = lax.fori_loop(0, 31, step, (lo, hi))
        ans = hi
    else:
        lo = jnp.zeros((rows, 1), jnp.int32)
        hi = jnp.full((rows, 1), 0x7F800000, jnp.int32)

        def step(_, carry):
            lo, hi = carry
            mid = lo + ((hi - lo) >> 1)
            cnt = jnp.sum((bits >= mid).astype(F32), axis=1, keepdims=True)
            pred = cnt >= kf
            return jnp.where(pred, mid, lo), jnp.where(pred, hi, mid)

        lo, hi = lax.fori_loop(0, 31, step, (lo, hi))
        ans = lo
    return lax.bitcast_convert_type(ans, F32)


def _d2_panel(xb, xf):
    sqb = jnp.sum(xb * xb, axis=1, keepdims=True)
    sqf = jnp.sum(xf * xf, axis=1)
    panel = _dg(xb, xf, ((1,), (1,)))
    return jnp.maximum(sqb + sqf[None, :] - 2.0 * panel, 0.0)


def _col_to_row(col):
    n = col.shape[0]
    r = lax.broadcasted_iota(jnp.int32, (n, n), 0)
    c = lax.broadcasted_iota(jnp.int32, (n, n), 1)
    eye = (r == c).astype(F32)
    return _dg_hi(col, eye, ((0,), (0,)))


def _row_to_col(row):
    n = row.shape[1]
    r = lax.broadcasted_iota(jnp.int32, (n, n), 0)
    c = lax.broadcasted_iota(jnp.int32, (n, n), 1)
    eye = (r == c).astype(F32)
    return _dg_hi(eye, row, ((0,), (1,)))


def _scale_body(x_ref, xf_ref, sc_ref, sr_ref):
    d2 = _d2_panel(x_ref[...], xf_ref[...])
    t = _kth_extract(d2, SCALE_K + 1, "min")
    s = jnp.sqrt(t + 1e-12)
    sc_ref[...] = s
    sr_ref[...] = _col_to_row(s)


def _scale_kernel(x):
    return pl.pallas_call(
        _scale_body,
        grid=(N // BMS,),
        in_specs=[
            pl.BlockSpec((BMS, D_FEAT), lambda i: (i, 0)),
            pl.BlockSpec((N, D_FEAT), lambda i: (0, 0)),
        ],
        out_specs=[
            pl.BlockSpec((BMS, 1), lambda i: (i, 0)),
            pl.BlockSpec((1, BMS), lambda i: (0, i)),
        ],
        out_shape=[
            jax.ShapeDtypeStruct((N, 1), F32),
            jax.ShapeDtypeStruct((1, N), F32),
        ],
    )(x, x)


def _wm_body(x_ref, xf_ref, sc_ref, sr_ref, wm_ref, rs_ref, cs_ref):
    i = pl.program_id(0)
    d2 = _d2_panel(x_ref[...], xf_ref[...])
    wm = jnp.exp(-d2 / (sc_ref[...] * sr_ref[...] + 1e-8))
    thr = _kth_extract(wm, N_NEIGHBORS + 1, "max")
    wmm = wm * (wm >= thr).astype(F32)
    wm_ref[...] = wmm.astype(jnp.bfloat16)
    rs = jnp.sum(wmm, axis=1, keepdims=True)
    rs_ref[...] = _col_to_row(rs)

    @pl.when(i == 0)
    def _():
        cs_ref[...] = jnp.zeros_like(cs_ref)

    cs_ref[...] += jnp.sum(wmm, axis=0, keepdims=True)


def _wm_kernel(x, scale_col, scale_row):
    return pl.pallas_call(
        _wm_body,
        grid=(N // BMS,),
        in_specs=[
            pl.BlockSpec((BMS, D_FEAT), lambda i: (i, 0)),
            pl.BlockSpec((N, D_FEAT), lambda i: (0, 0)),
            pl.BlockSpec((BMS, 1), lambda i: (i, 0)),
            pl.BlockSpec((1, N), lambda i: (0, 0)),
        ],
        out_specs=[
            pl.BlockSpec((BMS, N), lambda i: (i, 0)),
            pl.BlockSpec((1, BMS), lambda i: (0, i)),
            pl.BlockSpec((1, N), lambda i: (0, 0)),
        ],
        out_shape=[
            jax.ShapeDtypeStruct((N, N), jnp.bfloat16),
            jax.ShapeDtypeStruct((1, N), F32),
            jax.ShapeDtypeStruct((1, N), F32),
        ],
    )(x, x, scale_col, scale_row)


def _prep_body(rs_ref, cs_ref, xg_ref, dinv_ref, c_ref):
    i = pl.program_id(0)
    drow = 0.5 * (rs_ref[...] + cs_ref[...])
    dinv = 1.0 / jnp.sqrt(drow + 1e-8)
    dinv_ref[...] = dinv
    c_ref[...] = xg_ref[...] * _row_to_col(dinv)


def _prep_kernel(rowsum_row, colsum_row, Xg1):
    return pl.pallas_call(
        _prep_body,
        grid=(N // BM,),
        in_specs=[
            pl.BlockSpec((1, BM), lambda i: (0, i)),
            pl.BlockSpec((1, BM), lambda i: (0, i)),
            pl.BlockSpec((BM, GCN_HID), lambda i: (i, 0)),
        ],
        out_specs=[
            pl.BlockSpec((1, BM), lambda i: (0, i)),
            pl.BlockSpec((BM, GCN_HID), lambda i: (i, 0)),
        ],
        out_shape=[
            jax.ShapeDtypeStruct((1, N), F32),
            jax.ShapeDtypeStruct((N, GCN_HID), F32),
        ],
    )(rowsum_row, colsum_row, Xg1)


def _gcn1_body(wr_ref, wc_ref, c_ref, dv_ref, wg2_ref, c2_ref):
    dcol = _row_to_col(dv_ref[...])
    t1 = _dot(wr_ref[...], c_ref[...])
    t2 = _dg(wc_ref[...], c_ref[...], ((0,), (0,)))
    h = jnp.maximum(0.5 * dcol * (t1 + t2), 0.0)
    c2_ref[...] = dcol * _dot(h, wg2_ref[...])


def _gcn1_kernel(Wmm, C, dinv_row, Wg2):
    return pl.pallas_call(
        _gcn1_body,
        grid=(N // BM,),
        in_specs=[
            pl.BlockSpec((BM, N), lambda i: (i, 0)),
            pl.BlockSpec((N, BM), lambda i: (0, i)),
            pl.BlockSpec((N, GCN_HID), lambda i: (0, 0)),
            pl.BlockSpec((1, BM), lambda i: (0, i)),
            pl.BlockSpec((GCN_HID, GCN_OUT), lambda i: (0, 0)),
        ],
        out_specs=pl.BlockSpec((BM, GCN_OUT), lambda i: (i, 0)),
        out_shape=jax.ShapeDtypeStruct((N, GCN_OUT), F32),
    )(Wmm, Wmm, C, dinv_row, Wg2)


def _gcn2_body(wr_ref, wc_ref, c2_ref, dv_ref, out_ref):
    dcol = _row_to_col(dv_ref[...])
    t1 = _dot(wr_ref[...], c2_ref[...])
    t2 = _dg(wc_ref[...], c2_ref[...], ((0,), (0,)))
    out_ref[...] = 0.5 * dcol * (t1 + t2)


def _gcn2_kernel(Wmm, C2, dinv_row):
    return pl.pallas_call(
        _gcn2_body,
        grid=(N // BM,),
        in_specs=[
            pl.BlockSpec((BM, N), lambda i: (i, 0)),
            pl.BlockSpec((N, BM), lambda i: (0, i)),
            pl.BlockSpec((N, GCN_OUT), lambda i: (0, 0)),
            pl.BlockSpec((1, BM), lambda i: (0, i)),
        ],
        out_specs=pl.BlockSpec((BM, GCN_OUT), lambda i: (i, 0)),
        out_shape=jax.ShapeDtypeStruct((N, GCN_OUT), F32),
    )(Wmm, Wmm, C2, dinv_row)


ROWS_PW = N // 32
CH = 8
NCHUNK = ROWS_PW // CH


def _sc_sqrt(x):
    i = plsc.bitcast(x, jnp.int32)
    r = plsc.bitcast(jnp.int32(0x5F3759DF) - (i >> 1), F32)
    for _ in range(3):
        r = r * (1.5 - 0.5 * x * r * r)
    s = x * r
    return 0.5 * (s + x / s)


def _sc_knn_body(os_hbm, idx_hbm, idxf_hbm, bet_hbm, alp_hbm,
                 a_hbm, eh_hbm,
                 idx_v, idx_f, selfo, g_os, stripe, ebuf, ib,
                 scal, sem_g, sem_s):
    wid = lax.axis_index("s") * 2 + lax.axis_index("c")
    base = wid * ROWS_PW

    pltpu.sync_copy(idx_hbm.at[pl.ds(base, ROWS_PW)], idx_v)
    pltpu.sync_copy(idxf_hbm.at[pl.ds(base * (K + 6), ROWS_PW * (K + 6))],
                    idx_f)
    pltpu.sync_copy(bet_hbm, scal)
    betav = scal[...]
    pltpu.sync_copy(alp_hbm, scal)
    alpav = scal[...]

    ii = lax.broadcasted_iota(jnp.int32, (16,), 0)
    valid = (ii >= 1) & (ii <= K)
    zeros16 = jnp.zeros((16,), F32)
    jf = (ii + 1).astype(F32)

    def zstep(t, _):
        stripe[t // (N // 16), pl.ds((t % (N // 16)) * 16, 16)] = zeros16
        return 0
    lax.fori_loop(0, CH * (N // 16), zstep, 0)

    def chunk_step(c, _):
        row0 = base + c * CH
        gcp = pltpu.async_copy(os_hbm.at[idx_f.at[pl.ds(c * CH * 16, CH * 16)]],
                               g_os, sem_g)
        pltpu.sync_copy(os_hbm.at[pl.ds(row0, CH)], selfo)
        gcp.wait()

        def row_step(r, _):
            i0 = c * CH + r
            idxvec = idx_v[i0]
            d2 = zeros16
            for j in range(1, K + 1):
                acc = zeros16
                for q in range(4):
                    t = g_os[r * 16 + j, pl.ds(q * 16, 16)] - selfo[r, pl.ds(q * 16, 16)]
                    acc = acc + t * t
                d2 = jnp.where(ii == j, jnp.sum(acc), d2)
            dxi = _sc_sqrt(d2 + 1e-8)
            dfi = _sc_sqrt(4096.0 * d2 + 1e-12)
            ad = -(dxi + betav * dfi) / (2.0 * alpav)
            adm = jnp.where(valid, ad, -3.0e38)
            u, _unused = plsc.sort_key_val(adm, adm, descending=True)
            css = plsc.cumsum(u)
            cond = ((u + (1.0 - css) / jf) > 0) & (ii < K)
            rho = plsc.all_reduce_population_count(cond)
            rhof = rho.astype(F32)
            cssrho = jnp.sum(jnp.where(ii == rho - 1, css, 0.0))
            theta = (cssrho - 1.0) / rhof
            vals = jnp.where(valid, jnp.maximum(ad - theta, 0.0), 0.0)
            ib[...] = idxvec
            dup = ii < 0
            for s in range(1, K):
                sh = plsc.load_gather(ib, [jnp.minimum(ii + s, 15)])
                dup = dup | ((idxvec == sh) & (ii >= 1) & (ii + s <= K))
            wvals = jnp.where(valid & (~dup), vals, 0.0)
            rsplat = jnp.broadcast_to(r, (16,)).astype(jnp.int32)
            plsc.store_scatter(stripe, [rsplat, idxvec], vals, mask=valid)
            ib[...] = plsc.bitcast(wvals, jnp.int32)
            accs = [zeros16] * 4
            for j in range(1, K + 1):
                wj = plsc.bitcast(
                    plsc.load_gather(ib, [jnp.full((16,), j, jnp.int32)]), F32)
                for q in range(4):
                    accs[q] = accs[q] + wj * g_os[r * 16 + j, pl.ds(64 + q * 16, 16)]
            for q in range(4):
                ebuf[i0, pl.ds(q * 16, 16)] = accs[q]
            return 0

        lax.fori_loop(0, CH, row_step, 0)
        pltpu.sync_copy(stripe, a_hbm.at[pl.ds(row0, CH)])

        def rz_step(r, _):
            idxvec = idx_v[c * CH + r]
            rsplat = jnp.broadcast_to(r, (16,)).astype(jnp.int32)
            plsc.store_scatter(stripe, [rsplat, idxvec], zeros16, mask=valid)
            return 0
        lax.fori_loop(0, CH, rz_step, 0)
        return 0

    lax.fori_loop(0, NCHUNK, chunk_step, 0)
    pltpu.sync_copy(ebuf, eh_hbm.at[pl.ds(base, ROWS_PW)])


def _sc_knn(OS, idx, beta16, alpha16):
    mesh = plsc.VectorSubcoreMesh(core_axis_name="c", subcore_axis_name="s")
    f = functools.partial(
        pl.kernel,
        out_type=[
            jax.ShapeDtypeStruct((N, N), F32),
            jax.ShapeDtypeStruct((N, OUT_FEAT), F32),
        ],
        mesh=mesh,
        compiler_params=pltpu.CompilerParams(needs_layout_passes=False),
        scratch_types=[
            pltpu.VMEM((ROWS_PW, K + 6), jnp.int32),
            pltpu.VMEM((ROWS_PW * (K + 6),), jnp.int32),
            pltpu.VMEM((CH, 2 * OUT_FEAT), F32),
            pltpu.VMEM((CH * 16, 2 * OUT_FEAT), F32),
            pltpu.VMEM((CH, N), F32),
            pltpu.VMEM((ROWS_PW, OUT_FEAT), F32),
            pltpu.VMEM((16,), jnp.int32),
            pltpu.VMEM((16,), F32),
            pltpu.SemaphoreType.DMA,
            pltpu.SemaphoreType.DMA,
        ],
    )(_sc_knn_body)
    idx_flat = idx.reshape(N * (K + 6))
    return f(OS, idx, idx_flat, beta16, alpha16)


def kernel(x, x_orth, beta, alpha, idx, W1, b1, W2, b2, Wg1, Wg2):
    b1r = b1.reshape(1, HID)
    b2r = b2.reshape(1, OUT_FEAT)
    idx = idx.astype(jnp.int32)
    beta16 = jnp.full((16,), beta, F32)
    alpha16 = jnp.full((16,), alpha, F32)

    G = _gram(x_orth, W1, b1r, W2, b2r)
    Linv = _chol_inv(G)
    OS, Y, Xg1 = _mlp_all(x, W1, b1r, W2, b2r, Linv, Wg1)

    scale_col, scale_row = _scale_kernel(x)
    Wmm, rowsum_row, colsum_row = _wm_kernel(x, scale_col, scale_row)
    dinv_row, C = _prep_kernel(rowsum_row, colsum_row, Xg1)
    C2 = _gcn1_kernel(Wmm, C, dinv_row, Wg2)
    embs_graph = _gcn2_kernel(Wmm, C2, dinv_row)

    A, embs_hom = _sc_knn(OS, idx, beta16, alpha16)
    return (embs_hom, embs_graph, A, Y)
